# baseline (speedup 1.0000x reference)
import functools

import jax
import jax.numpy as jnp
from jax import lax
from jax.experimental import pallas as pl
from jax.experimental.pallas import tpu as pltpu
from jax.experimental.pallas import tpu_sc as plsc

N_NODES = {"movie": 10000, "user": 10000, "genre": 500, "conversation": 5000}
RELS = [
    ("has_genre", "movie", "genre", 30000),
    ("has_movie", "genre", "movie", 30000),
    ("rated_high", "user", "movie", 160000),
    ("rated_by", "movie", "user", 160000),
    ("mentions", "conversation", "movie", 25000),
    ("mentioned_in", "movie", "conversation", 25000),
]
LAYERS = [(128, 256, 2, True), (512, 256, 2, True), (512, 384, 1, False)]
TYPES = ["movie", "user", "genre", "conversation"]
GROUPS = [["movie", "genre"], ["user", "conversation"]]

NT = 32
CH = 128


def _ru(x, m):
    return -(-x // m) * m


RNG = {t: _ru(-(-(n + 1) // NT), 8) for t, n in N_NODES.items()}
NOUT = {t: NT * r for t, r in RNG.items()}
RP1 = {t: _ru(r + 1, 8) for t, r in RNG.items()}
CAP = {"has_genre": 1280, "has_movie": 1280, "rated_high": 6144,
       "rated_by": 6144, "mentions": 1152, "mentioned_in": 1152}
E128 = {name: _ru(e, CH) for name, _, _, e in RELS}
ASIZE = 20608


def _mesh():
    return plsc.VectorSubcoreMesh(core_axis_name="c", subcore_axis_name="s",
                                  num_cores=2, num_subcores=16)


def _cparams():
    return pltpu.CompilerParams(needs_layout_passes=False)


def _den_layout(h):
    bases, off = {}, 0
    for name, _, d, _ in RELS:
        bases[name] = off
        off += _ru(NT * h * RP1[d], 128)
    return bases, off


def _wid():
    return lax.axis_index("c") * 16 + lax.axis_index("s")


def _count(cref, wid, buf):
    pltpu.sync_copy(cref.at[pl.ds(wid * 16, 16)], buf)
    cv = buf[pl.ds(0, 16)]
    return (cv[0] + CH - 1) // CH


def _make_k1():
    out_type = []
    for name, _s, _d, _e in RELS:
        c = CAP[name]
        out_type += [jax.ShapeDtypeStruct((NT * c,), jnp.int32),
                     jax.ShapeDtypeStruct((NT * c,), jnp.int32),
                     jax.ShapeDtypeStruct((NT * 16,), jnp.int32)]
    scratch = [
        pltpu.VMEM((6144,), jnp.int32),
        pltpu.VMEM((6144,), jnp.int32),
        pltpu.VMEM((CH,), jnp.int32),
        pltpu.VMEM((CH,), jnp.int32),
        pltpu.VMEM((16,), jnp.int32),
    ]

    @functools.partial(pl.kernel, out_type=tuple(out_type), mesh=_mesh(),
                       scratch_types=scratch, compiler_params=_cparams(),
                       name="gat_bucket_edges")
    def k1(*refs):
        ins = refs[:12]
        outs = refs[12:12 + 18]
        bs, bd, sbuf, dbuf, cbuf = refs[30:]
        wid = _wid()
        for ri, (name, _st, dt, _e) in enumerate(RELS):
            srcr, dstr = ins[ri], ins[6 + ri]
            bsrc_o, bdst_o, cnt_o = outs[3 * ri], outs[3 * ri + 1], outs[3 * ri + 2]
            cap = CAP[name]
            rng = RNG[dt]
            lo = wid * rng

            @pl.loop(0, cap // 16)
            def _fill(i, _rng=rng):
                bs[pl.ds(i * 16, 16)] = jnp.zeros((16,), jnp.int32)
                bd[pl.ds(i * 16, 16)] = jnp.broadcast_to(
                    jnp.int32(_rng), (16,))

            @pl.loop(0, E128[name] // CH, init_carry=jnp.int32(0))
            def _scan(c, cnt, _srcr=srcr, _dstr=dstr, _lo=lo, _rng=rng):
                pltpu.sync_copy(_srcr.at[pl.ds(c * CH, CH)], sbuf)
                pltpu.sync_copy(_dstr.at[pl.ds(c * CH, CH)], dbuf)
                for j in range(CH // 16):
                    s16 = sbuf[pl.ds(j * 16, 16)]
                    d16 = dbuf[pl.ds(j * 16, 16)]
                    m = (d16 >= _lo) & (d16 < _lo + _rng)
                    plsc.store_compressed(bs.at[pl.ds(cnt, 16)], s16, mask=m)
                    plsc.store_compressed(bd.at[pl.ds(cnt, 16)], d16 - _lo,
                                          mask=m)
                    pc = plsc.all_reduce_population_count(m)
                    cnt = cnt + pc[0]
                return cnt

            cnt = _scan
            cbuf[pl.ds(0, 16)] = jnp.broadcast_to(cnt, (16,))
            pltpu.sync_copy(cbuf, cnt_o.at[pl.ds(wid * 16, 16)])
            pltpu.sync_copy(bs.at[pl.ds(0, cap)],
                            bsrc_o.at[pl.ds(wid * cap, cap)])
            pltpu.sync_copy(bd.at[pl.ds(0, cap)],
                            bdst_o.at[pl.ds(wid * cap, cap)])

    return k1


def _make_k2(h, bks, suffix):
    dbases, dtot = _den_layout(h)
    out_type = tuple(
        jax.ShapeDtypeStruct((h * NT * CAP[name],), jnp.float32)
        for name, _, _, _ in RELS
    ) + (jax.ShapeDtypeStruct((dtot,), jnp.float32),)
    scratch = [
        pltpu.VMEM((ASIZE,), jnp.float32),
        pltpu.VMEM((ASIZE,), jnp.float32),
        pltpu.VMEM((656,), jnp.float32),
        pltpu.VMEM((CH,), jnp.int32),
        pltpu.VMEM((CH,), jnp.int32),
        pltpu.VMEM((2 * CH,), jnp.float32),
        pltpu.VMEM((16,), jnp.int32),
    ]

    @functools.partial(pl.kernel, out_type=out_type, mesh=_mesh(),
                       scratch_types=scratch, compiler_params=_cparams(),
                       name="gat_edge_softmax_" + suffix)
    def k2(*refs):
        ins = refs[:30]
        exouts = refs[30:36]
        den = refs[36]
        asb, adb, slab, sbuf, dbuf, exbuf, cbuf = refs[37:]
        wid = _wid()
        for ri, (name, st, dt, _e) in enumerate(RELS):
            bsrc, bdst, cnts = ins[3 * ri], ins[3 * ri + 1], ins[3 * ri + 2]
            asr, adr = ins[18 + ri], ins[24 + ri]
            cap = CAP[name]
            rng, rp1 = RNG[dt], RP1[dt]
            lo = wid * rng
            hrp = h * rp1
            exout = exouts[ri]
            pltpu.sync_copy(asr, asb.at[pl.ds(0, N_NODES[st] * h)])
            pltpu.sync_copy(adr, adb.at[pl.ds(0, N_NODES[dt] * h)])

            @pl.loop(0, _ru(hrp, 16) // 16)
            def _zero(i):
                slab[pl.ds(i * 16, 16)] = jnp.zeros((16,), jnp.float32)

            nch = _count(cnts, wid, cbuf)

            @pl.loop(0, nch)
            def _chunk(c, _bsrc=bsrc, _bdst=bdst, _exout=exout, _cap=cap,
                       _lo=lo, _rp1=rp1):
                b = wid * _cap + c * CH
                pltpu.sync_copy(_bsrc.at[pl.ds(b, CH)], sbuf)
                pltpu.sync_copy(_bdst.at[pl.ds(b, CH)], dbuf)
                for j in range(CH // 16):
                    s16 = sbuf[pl.ds(j * 16, 16)]
                    d16 = dbuf[pl.ds(j * 16, 16)]
                    for hh in range(h):
                        av = plsc.load_gather(asb, [s16 * h + hh])
                        bv = plsc.load_gather(adb, [(d16 + _lo) * h + hh])
                        al = av + bv
                        al = jnp.maximum(al, al * 0.2)
                        ex = jnp.exp(al)
                        plsc.addupdate_scatter(slab, [d16 + hh * _rp1], ex)
                        exbuf[pl.ds(hh * CH + j * 16, 16)] = ex
                for hh in range(h):
                    pltpu.sync_copy(
                        exbuf.at[pl.ds(hh * CH, CH)],
                        _exout.at[pl.ds(hh * NT * _cap + b, CH)])

            pltpu.sync_copy(
                slab.at[pl.ds(0, hrp)],
                den.at[pl.ds(dbases[name] + wid * hrp, hrp)])

    def run(a_src, a_dst):
        outs = k2(*(
            [x for name, _, _, _ in RELS for x in bks[name]]
            + [a_src[r[0]] for r in RELS] + [a_dst[r[0]] for r in RELS]))
        return {r[0]: outs[i] for i, r in enumerate(RELS)}, outs[6]

    return run, dbases, dtot


def _rdenom(den):
    dtot = den.shape[0]

    def body(dref, oref):
        oref[...] = 1.0 / (dref[...] + 1e-16)

    return pl.pallas_call(
        body, out_shape=jax.ShapeDtypeStruct((1, dtot), jnp.float32))(
            den.reshape(1, dtot)).reshape(-1)


def _make_k3(h, chd, npass, p, group_rels, tbases, slab_rows, gtypes,
             dbases, suffix):
    head = (p * 128) // chd
    nrel = len(group_rels)
    out_type = tuple(jax.ShapeDtypeStruct((NOUT[t], 128), jnp.float32)
                     for t in gtypes)
    scratch = [
        pltpu.VMEM((slab_rows, 128), jnp.float32),
        pltpu.VMEM((CH, 128), jnp.float32),
        pltpu.VMEM((656,), jnp.float32),
        pltpu.VMEM((CH,), jnp.int32),
        pltpu.VMEM((CH,), jnp.int32),
        pltpu.VMEM((CH,), jnp.float32),
        pltpu.VMEM((CH,), jnp.int32),
        pltpu.VMEM((CH,), jnp.float32),
        pltpu.VMEM((16,), jnp.int32),
        pltpu.SemaphoreType.DMA,
    ]

    @functools.partial(pl.kernel, out_type=out_type, mesh=_mesh(),
                       scratch_types=scratch, compiler_params=_cparams(),
                       name="gat_aggregate_" + suffix)
    def k3(*refs):
        ins = refs[:3 * nrel]
        exs = refs[3 * nrel:4 * nrel]
        hss = refs[4 * nrel:5 * nrel]
        rdfl = refs[5 * nrel]
        zz = refs[5 * nrel + 1]
        outs = refs[5 * nrel + 2:5 * nrel + 2 + len(gtypes)]
        slab, rows, rdb, sbuf, dbuf, ebuf, gix, abuf, cbuf, sem = refs[
            5 * nrel + 2 + len(gtypes):]
        wid = _wid()

        for i in range(slab_rows // CH):
            pltpu.sync_copy(zz, slab.at[pl.ds(i * CH, CH)])
        rem = slab_rows % CH
        if rem:
            pltpu.sync_copy(zz.at[pl.ds(0, rem)],
                            slab.at[pl.ds(slab_rows - rem, rem)])

        coli = [jnp.arange(j * 16, j * 16 + 16, dtype=jnp.int32)
                for j in range(CH // 16)]

        for ri, (name, st, dt, _e) in enumerate(group_rels):
            bsrc, bdst, cnts = ins[3 * ri], ins[3 * ri + 1], ins[3 * ri + 2]
            cap = CAP[name]
            rp1 = RP1[dt]
            tb = tbases[dt]
            pltpu.sync_copy(
                rdfl.at[pl.ds(dbases[name] + wid * h * rp1 + head * rp1,
                              rp1)],
                rdb.at[pl.ds(0, rp1)])
            nch = _count(cnts, wid, cbuf)

            @pl.loop(0, nch)
            def _chunk(c, _bsrc=bsrc, _bdst=bdst, _ex=exs[ri], _hs=hss[ri],
                       _cap=cap, _tb=tb):
                b = wid * _cap + c * CH
                pltpu.sync_copy(_bsrc.at[pl.ds(b, CH)], sbuf)
                pltpu.sync_copy(_bdst.at[pl.ds(b, CH)], dbuf)
                pltpu.sync_copy(_ex.at[pl.ds(head * NT * _cap + b, CH)], ebuf)
                for j in range(CH // 16):
                    s16 = sbuf[pl.ds(j * 16, 16)]
                    d16 = dbuf[pl.ds(j * 16, 16)]
                    gix[pl.ds(j * 16, 16)] = s16 * npass + p
                    rdv = plsc.load_gather(rdb, [d16])
                    abuf[pl.ds(j * 16, 16)] = ebuf[pl.ds(j * 16, 16)] * rdv
                pltpu.async_copy(_hs.at[gix], rows, sem).wait()

                @pl.loop(0, CH)
                def _acc(k):
                    k16 = jnp.broadcast_to(k, (16,))
                    av = plsc.load_gather(abuf, [k16])
                    dlv = plsc.load_gather(dbuf, [k16]) + _tb
                    for j in range(CH // 16):
                        v = rows[k, pl.ds(j * 16, 16)] * av
                        plsc.addupdate_scatter(slab, [dlv, coli[j]], v)

        for ti, t in enumerate(gtypes):
            pltpu.sync_copy(
                slab.at[pl.ds(tbases[t], RNG[t])],
                outs[ti].at[pl.ds(wid * RNG[t], RNG[t])])

    return k3


def _proj(x, ws_list, wa, bm=512):
    n, f = x.shape
    nw = len(ws_list)
    grid = (pl.cdiv(n, bm),)

    def body(*refs):
        xr = refs[0]
        wrs = refs[1:1 + nw]
        war = refs[1 + nw]
        outs = refs[2 + nw:2 + 2 * nw]
        oa = refs[2 + 2 * nw]
        xv = xr[...]
        for wr, orf in zip(wrs, outs):
            orf[...] = jnp.dot(xv, wr[...], preferred_element_type=jnp.float32)
        oa[...] = jnp.dot(xv, war[...], preferred_element_type=jnp.float32)

    in_specs = ([pl.BlockSpec((bm, f), lambda i: (i, 0))]
                + [pl.BlockSpec((f, w.shape[1]), lambda i: (0, 0))
                   for w in ws_list]
                + [pl.BlockSpec((f, 128), lambda i: (0, 0))])
    out_specs = ([pl.BlockSpec((bm, w.shape[1]), lambda i: (i, 0))
                  for w in ws_list]
                 + [pl.BlockSpec((bm, 128), lambda i: (i, 0))])
    out_shape = ([jax.ShapeDtypeStruct((n, w.shape[1]), jnp.float32)
                  for w in ws_list]
                 + [jax.ShapeDtypeStruct((n, 128), jnp.float32)])
    return pl.pallas_call(body, grid=grid, in_specs=in_specs,
                          out_specs=out_specs, out_shape=out_shape)(
                              x, *ws_list, wa)


def _combine(parts, bias, n, width, relu, bm=512):
    npass = len(parts)

    def body(*refs):
        ins = refs[:npass]
        br = refs[npass]
        orf = refs[npass + 1]
        for p in range(npass):
            v = ins[p][...] + br[0, p * 128:(p + 1) * 128]
            orf[:, p * 128:(p + 1) * 128] = jnp.maximum(v, 0.0) if relu else v

    in_specs = ([pl.BlockSpec((bm, 128), lambda i: (i, 0))] * npass
                + [pl.BlockSpec((1, width), lambda i: (0, 0))])
    return pl.pallas_call(
        body, grid=(pl.cdiv(n, bm),), in_specs=in_specs,
        out_specs=pl.BlockSpec((bm, width), lambda i: (i, 0)),
        out_shape=jax.ShapeDtypeStruct((n, width), jnp.float32))(
            *parts, bias.reshape(1, width))


def kernel(x_movie, x_user, x_genre, x_conversation, params, ei_has_genre,
           ei_has_movie, ei_rated_high, ei_rated_by, ei_mentions,
           ei_mentioned_in):
    x = {"movie": x_movie, "user": x_user, "genre": x_genre,
         "conversation": x_conversation}
    ei = {"has_genre": ei_has_genre, "has_movie": ei_has_movie,
          "rated_high": ei_rated_high, "rated_by": ei_rated_by,
          "mentions": ei_mentions, "mentioned_in": ei_mentioned_in}

    srcp, dstp = [], []
    for name, _s, _d, e in RELS:
        pad = E128[name] - e
        srcp.append(jnp.concatenate([ei[name][0],
                                     jnp.zeros((pad,), jnp.int32)]))
        dstp.append(jnp.concatenate([ei[name][1],
                                     jnp.full((pad,), -1, jnp.int32)]))
    k1outs = _make_k1()(*(srcp + dstp))
    bks = {r[0]: k1outs[3 * i:3 * i + 3] for i, r in enumerate(RELS)}

    zz = jnp.zeros((CH, 128), jnp.float32)

    for l, (f_in, chd, h, concat) in enumerate(LAYERS):
        width = h * chd if concat else chd
        npass = width // 128
        lp = params[str(l)]

        wsrc, vsrc, vdst = {}, {}, {}
        for name, _s, _d, _e in RELS:
            pr = lp[name]
            wsrc[name] = pr["W_src"]
            vsrc[name] = jnp.einsum("fhc,hc->fh",
                                    pr["W_src"].reshape(f_in, h, chd),
                                    pr["att_src"])
            vdst[name] = jnp.einsum("fhc,hc->fh",
                                    pr["W_dst"].reshape(f_in, h, chd),
                                    pr["att_dst"])

        a_src, a_dst, hs = {}, {}, {}
        for t in TYPES:
            src_rels = [r for r in RELS if r[1] == t]
            dst_rels = [r for r in RELS if r[2] == t]
            ws_list = [wsrc[r[0]] for r in src_rels]
            acols = ([vsrc[r[0]] for r in src_rels]
                     + [vdst[r[0]] for r in dst_rels])
            na = sum(c.shape[1] for c in acols)
            wa = jnp.concatenate(
                acols + [jnp.zeros((f_in, 128 - na), jnp.float32)], axis=1)
            outs = _proj(x[t], ws_list, wa)
            for i, r in enumerate(src_rels):
                hs[r[0]] = outs[i]
            ac = outs[-1]
            off = 0
            for r in src_rels:
                a_src[r[0]] = ac[:, off:off + h].reshape(-1)
                off += h
            for r in dst_rels:
                a_dst[r[0]] = ac[:, off:off + h].reshape(-1)
                off += h

        k2run, dbases, _dtot = _make_k2(h, bks, f"l{l}")
        ex, den = k2run(a_src, a_dst)
        rden = _rdenom(den)

        outs_t = {}
        for g, gtypes in enumerate(GROUPS):
            rels_g = [r for r in RELS if r[2] in gtypes]
            tbases, off = {}, 0
            for t in gtypes:
                tbases[t] = off
                off += RP1[t]
            slab_rows = off
            accs = {t: [] for t in gtypes}
            for p in range(npass):
                k3 = _make_k3(h, chd, npass, p, rels_g, tbases, slab_rows,
                              gtypes, dbases, f"l{l}g{g}p{p}")
                o = k3(*([x for r in rels_g for x in bks[r[0]]]
                         + [ex[r[0]] for r in rels_g]
                         + [hs[r[0]].reshape(-1, 128) for r in rels_g]
                         + [rden, zz]))
                if len(gtypes) == 1:
                    o = (o,)
                for ti, t in enumerate(gtypes):
                    accs[t].append(o[ti])
            outs_t.update(accs)

        newx = {}
        for t in TYPES:
            bias_tot = sum(lp[r[0]]["bias"] for r in RELS if r[2] == t)
            newx[t] = _combine(outs_t[t], bias_tot, N_NODES[t], width,
                               relu=(l < len(LAYERS) - 1))
        x = newx

    return (x["movie"], x["user"], x["genre"], x["conversation"])


# speedup vs baseline: 1.1528x; 1.1528x over previous
import functools

import jax
import jax.numpy as jnp
from jax import lax
from jax.experimental import pallas as pl
from jax.experimental.pallas import tpu as pltpu
from jax.experimental.pallas import tpu_sc as plsc

N_NODES = {"movie": 10000, "user": 10000, "genre": 500, "conversation": 5000}
RELS = [
    ("has_genre", "movie", "genre", 30000),
    ("has_movie", "genre", "movie", 30000),
    ("rated_high", "user", "movie", 160000),
    ("rated_by", "movie", "user", 160000),
    ("mentions", "conversation", "movie", 25000),
    ("mentioned_in", "movie", "conversation", 25000),
]
LAYERS = [(128, 256, 2, True), (512, 256, 2, True), (512, 384, 1, False)]
TYPES = ["movie", "user", "genre", "conversation"]
GROUPS = [["movie", "genre"], ["user", "conversation"]]

NT = 32
CH = 128
SCAN = 4096


def _ru(x, m):
    return -(-x // m) * m


RNG = {t: _ru(-(-(n + 1) // NT), 8) for t, n in N_NODES.items()}
NOUT = {t: NT * r for t, r in RNG.items()}
RP1 = {t: _ru(r + 1, 8) for t, r in RNG.items()}
CAP = {"has_genre": 1280, "has_movie": 1280, "rated_high": 6144,
       "rated_by": 6144, "mentions": 1280, "mentioned_in": 1280}
E4K = {name: _ru(e, 2 * SCAN) for name, _, _, e in RELS}
ASIZE = 20608


def _mesh():
    return plsc.VectorSubcoreMesh(core_axis_name="c", subcore_axis_name="s",
                                  num_cores=2, num_subcores=16)


def _cparams():
    return pltpu.CompilerParams(needs_layout_passes=False)


def _wid():
    return lax.axis_index("c") * 16 + lax.axis_index("s")


def _count(cref, wid, buf):
    pltpu.sync_copy(cref.at[pl.ds(wid * 16, 16)], buf)
    cv = buf[pl.ds(0, 16)]
    return (cv[0] + CH - 1) // CH


def _make_k1():
    out_type = []
    for name, _s, _d, _e in RELS:
        c = CAP[name]
        out_type += [jax.ShapeDtypeStruct((NT * c,), jnp.int32),
                     jax.ShapeDtypeStruct((NT * c,), jnp.int32),
                     jax.ShapeDtypeStruct((NT * 16,), jnp.int32)]
    scratch = [
        pltpu.VMEM((6144,), jnp.int32),
        pltpu.VMEM((6144,), jnp.int32),
        pltpu.VMEM((SCAN,), jnp.int32), pltpu.VMEM((SCAN,), jnp.int32),
        pltpu.VMEM((SCAN,), jnp.int32), pltpu.VMEM((SCAN,), jnp.int32),
        pltpu.VMEM((16,), jnp.int32),
        pltpu.SemaphoreType.DMA, pltpu.SemaphoreType.DMA,
    ]

    @functools.partial(pl.kernel, out_type=tuple(out_type), mesh=_mesh(),
                       scratch_types=scratch, compiler_params=_cparams(),
                       name="gat_bucket_edges")
    def k1(*refs):
        ins = refs[:12]
        outs = refs[12:30]
        bs, bd, sb0, db0, sb1, db1, cbuf, sem0, sem1 = refs[30:]
        wid = _wid()
        for ri, (name, _st, dt, _e) in enumerate(RELS):
            srcr, dstr = ins[ri], ins[6 + ri]
            bsrc_o, bdst_o, cnt_o = (outs[3 * ri], outs[3 * ri + 1],
                                     outs[3 * ri + 2])
            cap = CAP[name]
            rng = RNG[dt]
            lo = wid * rng
            nck = E4K[name] // SCAN

            def _issue(c, sb, db, sem, _sr=srcr, _dr=dstr):
                pltpu.async_copy(_sr.at[pl.ds(c * SCAN, SCAN)], sb, sem)
                pltpu.async_copy(_dr.at[pl.ds(c * SCAN, SCAN)], db, sem)

            def _drain(sb, db, sem, _sr=srcr, _dr=dstr):
                pltpu.make_async_copy(_sr.at[pl.ds(0, SCAN)], sb, sem).wait()
                pltpu.make_async_copy(_dr.at[pl.ds(0, SCAN)], db, sem).wait()

            def _scan(sb, db, cnt, _lo=lo, _rng=rng):
                @pl.loop(0, SCAN // 16, init_carry=cnt)
                def _s(t, c2):
                    s16 = sb[pl.ds(t * 16, 16)]
                    d16 = db[pl.ds(t * 16, 16)]
                    m = (d16 >= _lo) & (d16 < _lo + _rng)
                    plsc.store_compressed(bs.at[pl.ds(c2, 16)], s16, mask=m)
                    plsc.store_compressed(bd.at[pl.ds(c2, 16)], d16 - _lo,
                                          mask=m)
                    return c2 + plsc.all_reduce_population_count(m)[0]
                return _s

            @pl.loop(0, cap // 16)
            def _fill(i, _rng=rng):
                bs[pl.ds(i * 16, 16)] = jnp.zeros((16,), jnp.int32)
                bd[pl.ds(i * 16, 16)] = jnp.broadcast_to(
                    jnp.int32(_rng), (16,))

            _issue(0, sb0, db0, sem0)

            @pl.loop(0, nck // 2, init_carry=jnp.int32(0))
            def _outer(i, cnt, _nck=nck):
                _issue(2 * i + 1, sb1, db1, sem1)
                _drain(sb0, db0, sem0)
                cnt = _scan(sb0, db0, cnt)
                _issue(jnp.minimum(2 * i + 2, _nck - 1), sb0, db0, sem0)
                _drain(sb1, db1, sem1)
                cnt = _scan(sb1, db1, cnt)
                return cnt

            cnt = _outer
            _drain(sb0, db0, sem0)
            cbuf[pl.ds(0, 16)] = jnp.broadcast_to(cnt, (16,))
            pltpu.sync_copy(cbuf, cnt_o.at[pl.ds(wid * 16, 16)])
            pltpu.sync_copy(bs.at[pl.ds(0, cap)],
                            bsrc_o.at[pl.ds(wid * cap, cap)])
            pltpu.sync_copy(bd.at[pl.ds(0, cap)],
                            bdst_o.at[pl.ds(wid * cap, cap)])

    return k1


def _make_k2(h, bks, suffix):
    out_type = tuple(
        jax.ShapeDtypeStruct((h * NT * CAP[name],), jnp.float32)
        for name, _, _, _ in RELS)
    scratch = [
        pltpu.VMEM((ASIZE,), jnp.float32),
        pltpu.VMEM((ASIZE,), jnp.float32),
        pltpu.VMEM((656,), jnp.float32),
        pltpu.VMEM((6144,), jnp.int32),
        pltpu.VMEM((6144,), jnp.int32),
        pltpu.VMEM((2 * 6144,), jnp.float32),
        pltpu.VMEM((16,), jnp.int32),
    ]

    @functools.partial(pl.kernel, out_type=out_type, mesh=_mesh(),
                       scratch_types=scratch, compiler_params=_cparams(),
                       name="gat_edge_softmax_" + suffix)
    def k2(*refs):
        ins = refs[:30]
        atouts = refs[30:36]
        asb, adb, slab, bsb, bdb, exb, cbuf = refs[36:]
        wid = _wid()
        for ri, (name, st, dt, _e) in enumerate(RELS):
            bsrc, bdst, cnts = ins[3 * ri], ins[3 * ri + 1], ins[3 * ri + 2]
            asr, adr = ins[18 + ri], ins[24 + ri]
            cap = CAP[name]
            rng, rp1 = RNG[dt], RP1[dt]
            lo = wid * rng
            hrp = h * rp1
            atout = atouts[ri]
            pltpu.sync_copy(asr, asb.at[pl.ds(0, N_NODES[st] * h)])
            pltpu.sync_copy(adr, adb.at[pl.ds(0, N_NODES[dt] * h)])
            pltpu.sync_copy(bsrc.at[pl.ds(wid * cap, cap)],
                            bsb.at[pl.ds(0, cap)])
            pltpu.sync_copy(bdst.at[pl.ds(wid * cap, cap)],
                            bdb.at[pl.ds(0, cap)])

            @pl.loop(0, _ru(hrp, 16) // 16)
            def _zero(i):
                slab[pl.ds(i * 16, 16)] = jnp.zeros((16,), jnp.float32)

            nch = _count(cnts, wid, cbuf)

            @pl.loop(0, nch)
            def _chunk(c, _cap=cap, _lo=lo, _rp1=rp1):
                for j in range(CH // 16):
                    b = c * CH + j * 16
                    s16 = bsb[pl.ds(b, 16)]
                    d16 = bdb[pl.ds(b, 16)]
                    for hh in range(h):
                        av = plsc.load_gather(asb, [s16 * h + hh])
                        bv = plsc.load_gather(adb, [(d16 + _lo) * h + hh])
                        al = av + bv
                        al = jnp.maximum(al, al * 0.2)
                        ex = jnp.exp(al)
                        plsc.addupdate_scatter(slab, [d16 + hh * _rp1], ex)
                        exb[pl.ds(hh * _cap + b, 16)] = ex

            @pl.loop(0, _ru(hrp, 16) // 16)
            def _inv(i):
                slab[pl.ds(i * 16, 16)] = 1.0 / (slab[pl.ds(i * 16, 16)]
                                                 + 1e-16)

            @pl.loop(0, nch)
            def _attn(c, _cap=cap, _rp1=rp1):
                for j in range(CH // 16):
                    b = c * CH + j * 16
                    d16 = bdb[pl.ds(b, 16)]
                    for hh in range(h):
                        rv = plsc.load_gather(slab, [d16 + hh * _rp1])
                        exb[pl.ds(hh * _cap + b, 16)] = (
                            exb[pl.ds(hh * _cap + b, 16)] * rv)

            for hh in range(h):
                pltpu.sync_copy(
                    exb.at[pl.ds(hh * cap, cap)],
                    atout.at[pl.ds(hh * NT * cap + wid * cap, cap)])

    def run(a_src, a_dst):
        outs = k2(*(
            [x for name, _, _, _ in RELS for x in bks[name]]
            + [a_src[r[0]] for r in RELS] + [a_dst[r[0]] for r in RELS]))
        return {r[0]: outs[i] for i, r in enumerate(RELS)}

    return run


def _make_k3(h, chd, npass, p, group_rels, tbases, slab_rows, gtypes,
             suffix):
    head = (p * 128) // chd
    nrel = len(group_rels)
    out_type = tuple(jax.ShapeDtypeStruct((NOUT[t], 128), jnp.float32)
                     for t in gtypes)
    scratch = [
        pltpu.VMEM((slab_rows, 128), jnp.float32),
        pltpu.VMEM((CH, 128), jnp.float32),
        pltpu.VMEM((CH, 128), jnp.float32),
        pltpu.VMEM((6144,), jnp.int32),
        pltpu.VMEM((6144,), jnp.int32),
        pltpu.VMEM((6144,), jnp.float32),
        pltpu.VMEM((CH,), jnp.int32),
        pltpu.VMEM((CH,), jnp.int32),
        pltpu.VMEM((16,), jnp.int32),
        pltpu.SemaphoreType.DMA, pltpu.SemaphoreType.DMA,
    ]

    @functools.partial(pl.kernel, out_type=out_type, mesh=_mesh(),
                       scratch_types=scratch, compiler_params=_cparams(),
                       name="gat_aggregate_" + suffix)
    def k3(*refs):
        ins = refs[:3 * nrel]
        ats = refs[3 * nrel:4 * nrel]
        hss = refs[4 * nrel:5 * nrel]
        zz = refs[5 * nrel]
        outs = refs[5 * nrel + 1:5 * nrel + 1 + len(gtypes)]
        (slab, rows0, rows1, bsb, bdb, atb, gix0, gix1, cbuf,
         sem0, sem1) = refs[5 * nrel + 1 + len(gtypes):]
        wid = _wid()

        for i in range(slab_rows // CH):
            pltpu.sync_copy(zz, slab.at[pl.ds(i * CH, CH)])
        rem = slab_rows % CH
        if rem:
            pltpu.sync_copy(zz.at[pl.ds(0, rem)],
                            slab.at[pl.ds(slab_rows - rem, rem)])

        coli = [jnp.arange(j * 16, j * 16 + 16, dtype=jnp.int32)
                for j in range(CH // 16)]

        for ri, (name, st, dt, _e) in enumerate(group_rels):
            bsrc, bdst, cnts = ins[3 * ri], ins[3 * ri + 1], ins[3 * ri + 2]
            cap = CAP[name]
            tb = tbases[dt]
            hsr = hss[ri]
            pltpu.sync_copy(bsrc.at[pl.ds(wid * cap, cap)],
                            bsb.at[pl.ds(0, cap)])
            pltpu.sync_copy(bdst.at[pl.ds(wid * cap, cap)],
                            bdb.at[pl.ds(0, cap)])
            pltpu.sync_copy(
                ats[ri].at[pl.ds(head * NT * cap + wid * cap, cap)],
                atb.at[pl.ds(0, cap)])
            nch = _count(cnts, wid, cbuf)

            def _prep(c, gix, rows, sem, _hs=hsr):
                base = c * CH
                for j in range(CH // 16):
                    s16 = bsb[pl.ds(base + j * 16, 16)]
                    gix[pl.ds(j * 16, 16)] = s16 * npass + p
                pltpu.async_copy(_hs.at[gix], rows, sem)

            def _acc_chunk(c, gix, rows, sem, _hs=hsr, _tb=tb):
                pltpu.make_async_copy(_hs.at[gix], rows, sem).wait()
                base = c * CH

                @pl.loop(0, CH, unroll=2)
                def _acc(k):
                    k16 = jnp.broadcast_to(base + k, (16,))
                    av = plsc.load_gather(atb, [k16])
                    dlv = plsc.load_gather(bdb, [k16]) + _tb
                    for j in range(CH // 16):
                        v = rows[k, pl.ds(j * 16, 16)] * av
                        plsc.addupdate_scatter(slab, [dlv, coli[j]], v)

            @pl.loop(0, (nch + 1) // 2)
            def _pair(i):
                _prep(2 * i, gix0, rows0, sem0)
                _prep(2 * i + 1, gix1, rows1, sem1)
                _acc_chunk(2 * i, gix0, rows0, sem0)
                _acc_chunk(2 * i + 1, gix1, rows1, sem1)

        for ti, t in enumerate(gtypes):
            pltpu.sync_copy(
                slab.at[pl.ds(tbases[t], RNG[t])],
                outs[ti].at[pl.ds(wid * RNG[t], RNG[t])])

    return k3


def _proj(x, ws_list, wa, bm=512):
    n, f = x.shape
    nw = len(ws_list)
    grid = (pl.cdiv(n, bm),)

    def body(*refs):
        xr = refs[0]
        wrs = refs[1:1 + nw]
        war = refs[1 + nw]
        outs = refs[2 + nw:2 + 2 * nw]
        oa = refs[2 + 2 * nw]
        xv = xr[...]
        for wr, orf in zip(wrs, outs):
            orf[...] = jnp.dot(xv, wr[...], preferred_element_type=jnp.float32)
        oa[...] = jnp.dot(xv, war[...], preferred_element_type=jnp.float32)

    in_specs = ([pl.BlockSpec((bm, f), lambda i: (i, 0))]
                + [pl.BlockSpec((f, w.shape[1]), lambda i: (0, 0))
                   for w in ws_list]
                + [pl.BlockSpec((f, 128), lambda i: (0, 0))])
    out_specs = ([pl.BlockSpec((bm, w.shape[1]), lambda i: (i, 0))
                  for w in ws_list]
                 + [pl.BlockSpec((bm, 128), lambda i: (i, 0))])
    out_shape = ([jax.ShapeDtypeStruct((n, w.shape[1]), jnp.float32)
                  for w in ws_list]
                 + [jax.ShapeDtypeStruct((n, 128), jnp.float32)])
    return pl.pallas_call(body, grid=grid, in_specs=in_specs,
                          out_specs=out_specs, out_shape=out_shape)(
                              x, *ws_list, wa)


def _combine(parts, bias, n, width, relu, bm=512):
    npass = len(parts)

    def body(*refs):
        ins = refs[:npass]
        br = refs[npass]
        orf = refs[npass + 1]
        for p in range(npass):
            v = ins[p][...] + br[0, p * 128:(p + 1) * 128]
            orf[:, p * 128:(p + 1) * 128] = jnp.maximum(v, 0.0) if relu else v

    in_specs = ([pl.BlockSpec((bm, 128), lambda i: (i, 0))] * npass
                + [pl.BlockSpec((1, width), lambda i: (0, 0))])
    return pl.pallas_call(
        body, grid=(pl.cdiv(n, bm),), in_specs=in_specs,
        out_specs=pl.BlockSpec((bm, width), lambda i: (i, 0)),
        out_shape=jax.ShapeDtypeStruct((n, width), jnp.float32))(
            *parts, bias.reshape(1, width))


def kernel(x_movie, x_user, x_genre, x_conversation, params, ei_has_genre,
           ei_has_movie, ei_rated_high, ei_rated_by, ei_mentions,
           ei_mentioned_in):
    x = {"movie": x_movie, "user": x_user, "genre": x_genre,
         "conversation": x_conversation}
    ei = {"has_genre": ei_has_genre, "has_movie": ei_has_movie,
          "rated_high": ei_rated_high, "rated_by": ei_rated_by,
          "mentions": ei_mentions, "mentioned_in": ei_mentioned_in}

    srcp, dstp = [], []
    for name, _s, _d, e in RELS:
        pad = E4K[name] - e
        srcp.append(jnp.concatenate([ei[name][0],
                                     jnp.zeros((pad,), jnp.int32)]))
        dstp.append(jnp.concatenate([ei[name][1],
                                     jnp.full((pad,), -1, jnp.int32)]))
    k1outs = _make_k1()(*(srcp + dstp))
    bks = {r[0]: k1outs[3 * i:3 * i + 3] for i, r in enumerate(RELS)}

    zz = jnp.zeros((CH, 128), jnp.float32)

    for l, (f_in, chd, h, concat) in enumerate(LAYERS):
        width = h * chd if concat else chd
        npass = width // 128
        lp = params[str(l)]

        wsrc, vsrc, vdst = {}, {}, {}
        for name, _s, _d, _e in RELS:
            pr = lp[name]
            wsrc[name] = pr["W_src"]
            vsrc[name] = jnp.einsum("fhc,hc->fh",
                                    pr["W_src"].reshape(f_in, h, chd),
                                    pr["att_src"])
            vdst[name] = jnp.einsum("fhc,hc->fh",
                                    pr["W_dst"].reshape(f_in, h, chd),
                                    pr["att_dst"])

        a_src, a_dst, hs = {}, {}, {}
        for t in TYPES:
            src_rels = [r for r in RELS if r[1] == t]
            dst_rels = [r for r in RELS if r[2] == t]
            ws_list = [wsrc[r[0]] for r in src_rels]
            acols = ([vsrc[r[0]] for r in src_rels]
                     + [vdst[r[0]] for r in dst_rels])
            na = sum(c.shape[1] for c in acols)
            wa = jnp.concatenate(
                acols + [jnp.zeros((f_in, 128 - na), jnp.float32)], axis=1)
            outs = _proj(x[t], ws_list, wa)
            for i, r in enumerate(src_rels):
                hs[r[0]] = outs[i]
            ac = outs[-1]
            off = 0
            for r in src_rels:
                a_src[r[0]] = ac[:, off:off + h].reshape(-1)
                off += h
            for r in dst_rels:
                a_dst[r[0]] = ac[:, off:off + h].reshape(-1)
                off += h

        attn = _make_k2(h, bks, f"l{l}")(a_src, a_dst)

        outs_t = {}
        for g, gtypes in enumerate(GROUPS):
            rels_g = [r for r in RELS if r[2] in gtypes]
            tbases, off = {}, 0
            for t in gtypes:
                tbases[t] = off
                off += RP1[t]
            slab_rows = off
            accs = {t: [] for t in gtypes}
            for p in range(npass):
                k3 = _make_k3(h, chd, npass, p, rels_g, tbases, slab_rows,
                              gtypes, f"l{l}g{g}p{p}")
                o = k3(*([x for r in rels_g for x in bks[r[0]]]
                         + [attn[r[0]] for r in rels_g]
                         + [hs[r[0]].reshape(-1, 128) for r in rels_g]
                         + [zz]))
                if not isinstance(o, (tuple, list)):
                    o = (o,)
                for ti, t in enumerate(gtypes):
                    accs[t].append(o[ti])
            outs_t.update(accs)

        newx = {}
        for t in TYPES:
            bias_tot = sum(lp[r[0]]["bias"] for r in RELS if r[2] == t)
            newx[t] = _combine(outs_t[t], bias_tot, N_NODES[t], width,
                               relu=(l < len(LAYERS) - 1))
        x = newx

    return (x["movie"], x["user"], x["genre"], x["conversation"])


# speedup vs baseline: 1.7696x; 1.5351x over previous
import functools

import jax
import jax.numpy as jnp
from jax import lax
from jax.experimental import pallas as pl
from jax.experimental.pallas import tpu as pltpu
from jax.experimental.pallas import tpu_sc as plsc

N_NODES = {"movie": 10000, "user": 10000, "genre": 500, "conversation": 5000}
RELS = [
    ("has_genre", "movie", "genre", 30000),
    ("has_movie", "genre", "movie", 30000),
    ("rated_high", "user", "movie", 160000),
    ("rated_by", "movie", "user", 160000),
    ("mentions", "conversation", "movie", 25000),
    ("mentioned_in", "movie", "conversation", 25000),
]
LAYERS = [(128, 256, 2, True), (512, 256, 2, True), (512, 384, 1, False)]
TYPES = ["movie", "user", "genre", "conversation"]
GROUPS = [["movie"], ["genre"], ["user"], ["conversation"]]

NT = 32
CH = 128
SCAN = 4096


def _ru(x, m):
    return -(-x // m) * m


RNG = {t: _ru(-(-(n + 1) // NT), 8) for t, n in N_NODES.items()}
NOUT = {t: NT * r for t, r in RNG.items()}
RP1 = {t: _ru(r + 1, 8) for t, r in RNG.items()}
CAP = {"has_genre": 1280, "has_movie": 1280, "rated_high": 6144,
       "rated_by": 6144, "mentions": 1280, "mentioned_in": 1280}
E4K = {name: _ru(e, 2 * SCAN) for name, _, _, e in RELS}
ASIZE = 20608


def _mesh():
    return plsc.VectorSubcoreMesh(core_axis_name="c", subcore_axis_name="s",
                                  num_cores=2, num_subcores=16)


def _cparams():
    return pltpu.CompilerParams(needs_layout_passes=False)


def _wid():
    return lax.axis_index("c") * 16 + lax.axis_index("s")


def _count(cref, wid, buf):
    pltpu.sync_copy(cref.at[pl.ds(wid * 16, 16)], buf)
    cv = buf[pl.ds(0, 16)]
    return (cv[0] + CH - 1) // CH


def _make_k1():
    out_type = []
    for name, _s, _d, _e in RELS:
        c = CAP[name]
        out_type += [jax.ShapeDtypeStruct((NT * c,), jnp.int32),
                     jax.ShapeDtypeStruct((NT * c,), jnp.int32),
                     jax.ShapeDtypeStruct((NT * 16,), jnp.int32)]
    scratch = [
        pltpu.VMEM((6144,), jnp.int32),
        pltpu.VMEM((6144,), jnp.int32),
        pltpu.VMEM((SCAN,), jnp.int32), pltpu.VMEM((SCAN,), jnp.int32),
        pltpu.VMEM((SCAN,), jnp.int32), pltpu.VMEM((SCAN,), jnp.int32),
        pltpu.VMEM((16,), jnp.int32),
        pltpu.SemaphoreType.DMA, pltpu.SemaphoreType.DMA,
    ]

    @functools.partial(pl.kernel, out_type=tuple(out_type), mesh=_mesh(),
                       scratch_types=scratch, compiler_params=_cparams(),
                       name="gat_bucket_edges")
    def k1(*refs):
        ins = refs[:12]
        outs = refs[12:30]
        bs, bd, sb0, db0, sb1, db1, cbuf, sem0, sem1 = refs[30:]
        wid = _wid()
        for ri, (name, _st, dt, _e) in enumerate(RELS):
            srcr, dstr = ins[ri], ins[6 + ri]
            bsrc_o, bdst_o, cnt_o = (outs[3 * ri], outs[3 * ri + 1],
                                     outs[3 * ri + 2])
            cap = CAP[name]
            rng = RNG[dt]
            lo = wid * rng
            nck = E4K[name] // SCAN

            def _issue(c, sb, db, sem, _sr=srcr, _dr=dstr):
                pltpu.async_copy(_sr.at[pl.ds(c * SCAN, SCAN)], sb, sem)
                pltpu.async_copy(_dr.at[pl.ds(c * SCAN, SCAN)], db, sem)

            def _drain(sb, db, sem, _sr=srcr, _dr=dstr):
                pltpu.make_async_copy(_sr.at[pl.ds(0, SCAN)], sb, sem).wait()
                pltpu.make_async_copy(_dr.at[pl.ds(0, SCAN)], db, sem).wait()

            def _scan(sb, db, cnt, _lo=lo, _rng=rng):
                @pl.loop(0, SCAN // 16, init_carry=cnt)
                def _s(t, c2):
                    s16 = sb[pl.ds(t * 16, 16)]
                    d16 = db[pl.ds(t * 16, 16)]
                    m = (d16 >= _lo) & (d16 < _lo + _rng)
                    plsc.store_compressed(bs.at[pl.ds(c2, 16)], s16, mask=m)
                    plsc.store_compressed(bd.at[pl.ds(c2, 16)], d16 - _lo,
                                          mask=m)
                    return c2 + plsc.all_reduce_population_count(m)[0]
                return _s

            @pl.loop(0, cap // 16)
            def _fill(i, _rng=rng):
                bs[pl.ds(i * 16, 16)] = jnp.zeros((16,), jnp.int32)
                bd[pl.ds(i * 16, 16)] = jnp.broadcast_to(
                    jnp.int32(_rng), (16,))

            _issue(0, sb0, db0, sem0)

            @pl.loop(0, nck // 2, init_carry=jnp.int32(0))
            def _outer(i, cnt, _nck=nck):
                _issue(2 * i + 1, sb1, db1, sem1)
                _drain(sb0, db0, sem0)
                cnt = _scan(sb0, db0, cnt)
                _issue(jnp.minimum(2 * i + 2, _nck - 1), sb0, db0, sem0)
                _drain(sb1, db1, sem1)
                cnt = _scan(sb1, db1, cnt)
                return cnt

            cnt = _outer
            _drain(sb0, db0, sem0)
            cbuf[pl.ds(0, 16)] = jnp.broadcast_to(cnt, (16,))
            pltpu.sync_copy(cbuf, cnt_o.at[pl.ds(wid * 16, 16)])
            pltpu.sync_copy(bs.at[pl.ds(0, cap)],
                            bsrc_o.at[pl.ds(wid * cap, cap)])
            pltpu.sync_copy(bd.at[pl.ds(0, cap)],
                            bdst_o.at[pl.ds(wid * cap, cap)])

    return k1


def _make_k2(h, bks, suffix):
    out_type = tuple(
        jax.ShapeDtypeStruct((h * NT * CAP[name],), jnp.float32)
        for name, _, _, _ in RELS)
    scratch = [
        pltpu.VMEM((ASIZE,), jnp.float32),
        pltpu.VMEM((ASIZE,), jnp.float32),
        pltpu.VMEM((656,), jnp.float32),
        pltpu.VMEM((6144,), jnp.int32),
        pltpu.VMEM((6144,), jnp.int32),
        pltpu.VMEM((2 * 6144,), jnp.float32),
        pltpu.VMEM((16,), jnp.int32),
    ]

    @functools.partial(pl.kernel, out_type=out_type, mesh=_mesh(),
                       scratch_types=scratch, compiler_params=_cparams(),
                       name="gat_edge_softmax_" + suffix)
    def k2(*refs):
        ins = refs[:30]
        atouts = refs[30:36]
        asb, adb, slab, bsb, bdb, exb, cbuf = refs[36:]
        wid = _wid()
        for ri, (name, st, dt, _e) in enumerate(RELS):
            bsrc, bdst, cnts = ins[3 * ri], ins[3 * ri + 1], ins[3 * ri + 2]
            asr, adr = ins[18 + ri], ins[24 + ri]
            cap = CAP[name]
            rng, rp1 = RNG[dt], RP1[dt]
            lo = wid * rng
            hrp = h * rp1
            atout = atouts[ri]
            pltpu.sync_copy(asr, asb.at[pl.ds(0, N_NODES[st] * h)])
            pltpu.sync_copy(adr, adb.at[pl.ds(0, N_NODES[dt] * h)])
            pltpu.sync_copy(bsrc.at[pl.ds(wid * cap, cap)],
                            bsb.at[pl.ds(0, cap)])
            pltpu.sync_copy(bdst.at[pl.ds(wid * cap, cap)],
                            bdb.at[pl.ds(0, cap)])

            @pl.loop(0, _ru(hrp, 16) // 16)
            def _zero(i):
                slab[pl.ds(i * 16, 16)] = jnp.zeros((16,), jnp.float32)

            nch = _count(cnts, wid, cbuf)

            @pl.loop(0, nch)
            def _chunk(c, _cap=cap, _lo=lo, _rp1=rp1):
                for j in range(CH // 16):
                    b = c * CH + j * 16
                    s16 = bsb[pl.ds(b, 16)]
                    d16 = bdb[pl.ds(b, 16)]
                    for hh in range(h):
                        av = plsc.load_gather(asb, [s16 * h + hh])
                        bv = plsc.load_gather(adb, [(d16 + _lo) * h + hh])
                        al = av + bv
                        al = jnp.maximum(al, al * 0.2)
                        ex = jnp.exp(al)
                        plsc.addupdate_scatter(slab, [d16 + hh * _rp1], ex)
                        exb[pl.ds(hh * _cap + b, 16)] = ex

            @pl.loop(0, _ru(hrp, 16) // 16)
            def _inv(i):
                slab[pl.ds(i * 16, 16)] = 1.0 / (slab[pl.ds(i * 16, 16)]
                                                 + 1e-16)

            @pl.loop(0, nch)
            def _attn(c, _cap=cap, _rp1=rp1):
                for j in range(CH // 16):
                    b = c * CH + j * 16
                    d16 = bdb[pl.ds(b, 16)]
                    for hh in range(h):
                        rv = plsc.load_gather(slab, [d16 + hh * _rp1])
                        exb[pl.ds(hh * _cap + b, 16)] = (
                            exb[pl.ds(hh * _cap + b, 16)] * rv)

            for hh in range(h):
                pltpu.sync_copy(
                    exb.at[pl.ds(hh * cap, cap)],
                    atout.at[pl.ds(hh * NT * cap + wid * cap, cap)])

    def run(a_src, a_dst):
        outs = k2(*(
            [x for name, _, _, _ in RELS for x in bks[name]]
            + [a_src[r[0]] for r in RELS] + [a_dst[r[0]] for r in RELS]))
        return {r[0]: outs[i] for i, r in enumerate(RELS)}

    return run


CH2 = 64


def _make_k3(h, chd, head, colw, group_rels, tbases, slab_rows, gtypes,
             suffix):
    nrel = len(group_rels)
    out_type = tuple(jax.ShapeDtypeStruct((NOUT[t], colw), jnp.float32)
                     for t in gtypes)
    scratch = [
        pltpu.VMEM((slab_rows, colw), jnp.float32),
        pltpu.VMEM((CH2, colw), jnp.float32),
        pltpu.VMEM((6144,), jnp.int32),
        pltpu.VMEM((6144,), jnp.int32),
        pltpu.VMEM((6144,), jnp.float32),
        pltpu.VMEM((CH2,), jnp.int32),
        pltpu.VMEM((16,), jnp.int32),
        pltpu.SemaphoreType.DMA,
    ]

    @functools.partial(pl.kernel, out_type=out_type, mesh=_mesh(),
                       scratch_types=scratch, compiler_params=_cparams(),
                       name="gat_aggregate_" + suffix)
    def k3(*refs):
        ins = refs[:3 * nrel]
        ats = refs[3 * nrel:4 * nrel]
        hss = refs[4 * nrel:5 * nrel]
        zz = refs[5 * nrel]
        outs = refs[5 * nrel + 1:5 * nrel + 1 + len(gtypes)]
        slab, rows, bsb, bdb, atb, gix, cbuf, sem = refs[
            5 * nrel + 1 + len(gtypes):]
        wid = _wid()

        for i in range(slab_rows // CH2):
            pltpu.sync_copy(zz, slab.at[pl.ds(i * CH2, CH2)])
        rem = slab_rows % CH2
        if rem:
            pltpu.sync_copy(zz.at[pl.ds(0, rem)],
                            slab.at[pl.ds(slab_rows - rem, rem)])

        coli = [jnp.arange(j * 16, j * 16 + 16, dtype=jnp.int32)
                for j in range(colw // 16)]

        for ri, (name, st, dt, _e) in enumerate(group_rels):
            bsrc, bdst, cnts = ins[3 * ri], ins[3 * ri + 1], ins[3 * ri + 2]
            cap = CAP[name]
            tb = tbases[dt]
            hsr = hss[ri]
            pltpu.sync_copy(bsrc.at[pl.ds(wid * cap, cap)],
                            bsb.at[pl.ds(0, cap)])
            pltpu.sync_copy(bdst.at[pl.ds(wid * cap, cap)],
                            bdb.at[pl.ds(0, cap)])
            pltpu.sync_copy(
                ats[ri].at[pl.ds(head * NT * cap + wid * cap, cap)],
                atb.at[pl.ds(0, cap)])
            pltpu.sync_copy(cnts.at[pl.ds(wid * 16, 16)], cbuf)
            cv = cbuf[pl.ds(0, 16)]
            nch = (cv[0] + CH2 - 1) // CH2

            @pl.loop(0, nch)
            def _chunk(c, _hs=hsr, _tb=tb):
                base = c * CH2
                for j in range(CH2 // 16):
                    s16 = bsb[pl.ds(base + j * 16, 16)]
                    gix[pl.ds(j * 16, 16)] = s16
                pltpu.async_copy(_hs.at[gix], rows, sem).wait()

                @pl.loop(0, CH2, unroll=2)
                def _acc(k):
                    k16 = jnp.broadcast_to(base + k, (16,))
                    av = plsc.load_gather(atb, [k16])
                    dlv = plsc.load_gather(bdb, [k16]) + _tb
                    for j in range(colw // 16):
                        v = rows[k, pl.ds(j * 16, 16)] * av
                        plsc.addupdate_scatter(slab, [dlv, coli[j]], v)

        for ti, t in enumerate(gtypes):
            pltpu.sync_copy(
                slab.at[pl.ds(tbases[t], RNG[t])],
                outs[ti].at[pl.ds(wid * RNG[t], RNG[t])])

    return k3


def _proj(x, ws_list, wa, bm=512):
    n, f = x.shape
    nw = len(ws_list)
    grid = (pl.cdiv(n, bm),)

    def body(*refs):
        xr = refs[0]
        wrs = refs[1:1 + nw]
        war = refs[1 + nw]
        outs = refs[2 + nw:2 + 2 * nw]
        oa = refs[2 + 2 * nw]
        xv = xr[...]
        for wr, orf in zip(wrs, outs):
            orf[...] = jnp.dot(xv, wr[...], preferred_element_type=jnp.float32)
        oa[...] = jnp.dot(xv, war[...], preferred_element_type=jnp.float32)

    in_specs = ([pl.BlockSpec((bm, f), lambda i: (i, 0))]
                + [pl.BlockSpec((f, w.shape[1]), lambda i: (0, 0))
                   for w in ws_list]
                + [pl.BlockSpec((f, 128), lambda i: (0, 0))])
    out_specs = ([pl.BlockSpec((bm, w.shape[1]), lambda i: (i, 0))
                  for w in ws_list]
                 + [pl.BlockSpec((bm, 128), lambda i: (i, 0))])
    out_shape = ([jax.ShapeDtypeStruct((n, w.shape[1]), jnp.float32)
                  for w in ws_list]
                 + [jax.ShapeDtypeStruct((n, 128), jnp.float32)])
    return pl.pallas_call(body, grid=grid, in_specs=in_specs,
                          out_specs=out_specs, out_shape=out_shape)(
                              x, *ws_list, wa)


def _combine(parts, bias, n, width, relu, bm=512):
    npass = len(parts)
    widths = [int(q.shape[1]) for q in parts]
    offs = [sum(widths[:i]) for i in range(npass)]

    def body(*refs):
        ins = refs[:npass]
        br = refs[npass]
        orf = refs[npass + 1]
        for p in range(npass):
            o, w = offs[p], widths[p]
            v = ins[p][...] + br[0, o:o + w]
            orf[:, o:o + w] = jnp.maximum(v, 0.0) if relu else v

    in_specs = ([pl.BlockSpec((bm, w), lambda i: (i, 0)) for w in widths]
                + [pl.BlockSpec((1, width), lambda i: (0, 0))])
    return pl.pallas_call(
        body, grid=(pl.cdiv(n, bm),), in_specs=in_specs,
        out_specs=pl.BlockSpec((bm, width), lambda i: (i, 0)),
        out_shape=jax.ShapeDtypeStruct((n, width), jnp.float32))(
            *parts, bias.reshape(1, width))


def kernel(x_movie, x_user, x_genre, x_conversation, params, ei_has_genre,
           ei_has_movie, ei_rated_high, ei_rated_by, ei_mentions,
           ei_mentioned_in):
    x = {"movie": x_movie, "user": x_user, "genre": x_genre,
         "conversation": x_conversation}
    ei = {"has_genre": ei_has_genre, "has_movie": ei_has_movie,
          "rated_high": ei_rated_high, "rated_by": ei_rated_by,
          "mentions": ei_mentions, "mentioned_in": ei_mentioned_in}

    srcp, dstp = [], []
    for name, _s, _d, e in RELS:
        pad = E4K[name] - e
        srcp.append(jnp.concatenate([ei[name][0],
                                     jnp.zeros((pad,), jnp.int32)]))
        dstp.append(jnp.concatenate([ei[name][1],
                                     jnp.full((pad,), -1, jnp.int32)]))
    k1outs = _make_k1()(*(srcp + dstp))
    bks = {r[0]: k1outs[3 * i:3 * i + 3] for i, r in enumerate(RELS)}

    for l, (f_in, chd, h, concat) in enumerate(LAYERS):
        width = h * chd if concat else chd
        npass = width // 128
        lp = params[str(l)]

        wsrc, vsrc, vdst = {}, {}, {}
        for name, _s, _d, _e in RELS:
            pr = lp[name]
            wsrc[name] = pr["W_src"]
            vsrc[name] = jnp.einsum("fhc,hc->fh",
                                    pr["W_src"].reshape(f_in, h, chd),
                                    pr["att_src"])
            vdst[name] = jnp.einsum("fhc,hc->fh",
                                    pr["W_dst"].reshape(f_in, h, chd),
                                    pr["att_dst"])

        segs = ([(0, 256), (256, width - 256)] if width > 256
                else [(0, width)])
        a_src, a_dst, hs = {}, {}, {}
        for t in TYPES:
            src_rels = [r for r in RELS if r[1] == t]
            dst_rels = [r for r in RELS if r[2] == t]
            ws_list = [wsrc[r[0]][:, c0:c0 + cw]
                       for r in src_rels for (c0, cw) in segs]
            acols = ([vsrc[r[0]] for r in src_rels]
                     + [vdst[r[0]] for r in dst_rels])
            na = sum(c.shape[1] for c in acols)
            wa = jnp.concatenate(
                acols + [jnp.zeros((f_in, 128 - na), jnp.float32)], axis=1)
            outs = _proj(x[t], ws_list, wa)
            oi = 0
            for r in src_rels:
                for si in range(len(segs)):
                    hs[(r[0], si)] = outs[oi]
                    oi += 1
            ac = outs[-1]
            off = 0
            for r in src_rels:
                a_src[r[0]] = ac[:, off:off + h].reshape(-1)
                off += h
            for r in dst_rels:
                a_dst[r[0]] = ac[:, off:off + h].reshape(-1)
                off += h

        attn = _make_k2(h, bks, f"l{l}")(a_src, a_dst)

        outs_t = {}
        for g, gtypes in enumerate(GROUPS):
            rels_g = [r for r in RELS if r[2] in gtypes]
            tbases, off = {}, 0
            for t in gtypes:
                tbases[t] = off
                off += RP1[t]
            slab_rows = off
            accs = {t: [] for t in gtypes}
            for si, (c0, cw) in enumerate(segs):
                head = c0 // chd
                k3 = _make_k3(h, chd, head, cw, rels_g, tbases, slab_rows,
                              gtypes, f"l{l}g{g}s{si}")
                o = k3(*([x for r in rels_g for x in bks[r[0]]]
                         + [attn[r[0]] for r in rels_g]
                         + [hs[(r[0], si)] for r in rels_g]
                         + [jnp.zeros((64, cw), jnp.float32)]))
                if not isinstance(o, (tuple, list)):
                    o = (o,)
                for ti, t in enumerate(gtypes):
                    accs[t].append(o[ti])
            outs_t.update(accs)

        newx = {}
        for t in TYPES:
            bias_tot = sum(lp[r[0]]["bias"] for r in RELS if r[2] == t)
            newx[t] = _combine(outs_t[t], bias_tot, N_NODES[t], width,
                               relu=(l < len(LAYERS) - 1))
        x = newx

    return (x["movie"], x["user"], x["genre"], x["conversation"])


# speedup vs baseline: 1.7968x; 1.0154x over previous
import functools

import jax
import jax.numpy as jnp
from jax import lax
from jax.experimental import pallas as pl
from jax.experimental.pallas import tpu as pltpu
from jax.experimental.pallas import tpu_sc as plsc

N_NODES = {"movie": 10000, "user": 10000, "genre": 500, "conversation": 5000}
RELS = [
    ("has_genre", "movie", "genre", 30000),
    ("has_movie", "genre", "movie", 30000),
    ("rated_high", "user", "movie", 160000),
    ("rated_by", "movie", "user", 160000),
    ("mentions", "conversation", "movie", 25000),
    ("mentioned_in", "movie", "conversation", 25000),
]
LAYERS = [(128, 256, 2, True), (512, 256, 2, True), (512, 384, 1, False)]
TYPES = ["movie", "user", "genre", "conversation"]
GROUPS = [["movie"], ["genre"], ["user"], ["conversation"]]

NT = 32
CH = 128
SCAN = 4096


def _ru(x, m):
    return -(-x // m) * m


RNG = {t: _ru(-(-(n + 1) // NT), 8) for t, n in N_NODES.items()}
NOUT = {t: NT * r for t, r in RNG.items()}
RP1 = {t: _ru(r + 1, 8) for t, r in RNG.items()}
CAP = {"has_genre": 1280, "has_movie": 1280, "rated_high": 6144,
       "rated_by": 6144, "mentions": 1280, "mentioned_in": 1280}
E4K = {name: _ru(e, 2 * SCAN) for name, _, _, e in RELS}
ASIZE = 20608


def _mesh():
    return plsc.VectorSubcoreMesh(core_axis_name="c", subcore_axis_name="s",
                                  num_cores=2, num_subcores=16)


def _cparams():
    return pltpu.CompilerParams(needs_layout_passes=False)


def _wid():
    return lax.axis_index("c") * 16 + lax.axis_index("s")


def _count(cref, wid, buf):
    pltpu.sync_copy(cref.at[pl.ds(wid * 16, 16)], buf)
    cv = buf[pl.ds(0, 16)]
    return (cv[0] + CH - 1) // CH


def _make_k1():
    out_type = []
    for name, _s, _d, _e in RELS:
        c = CAP[name]
        out_type += [jax.ShapeDtypeStruct((NT * c,), jnp.int32),
                     jax.ShapeDtypeStruct((NT * c,), jnp.int32),
                     jax.ShapeDtypeStruct((NT * 16,), jnp.int32)]
    scratch = [
        pltpu.VMEM((6144,), jnp.int32),
        pltpu.VMEM((6144,), jnp.int32),
        pltpu.VMEM((SCAN,), jnp.int32), pltpu.VMEM((SCAN,), jnp.int32),
        pltpu.VMEM((SCAN,), jnp.int32), pltpu.VMEM((SCAN,), jnp.int32),
        pltpu.VMEM((16,), jnp.int32),
        pltpu.SemaphoreType.DMA, pltpu.SemaphoreType.DMA,
    ]

    @functools.partial(pl.kernel, out_type=tuple(out_type), mesh=_mesh(),
                       scratch_types=scratch, compiler_params=_cparams(),
                       name="gat_bucket_edges")
    def k1(*refs):
        ins = refs[:12]
        outs = refs[12:30]
        bs, bd, sb0, db0, sb1, db1, cbuf, sem0, sem1 = refs[30:]
        wid = _wid()
        for ri, (name, _st, dt, _e) in enumerate(RELS):
            srcr, dstr = ins[ri], ins[6 + ri]
            bsrc_o, bdst_o, cnt_o = (outs[3 * ri], outs[3 * ri + 1],
                                     outs[3 * ri + 2])
            cap = CAP[name]
            rng = RNG[dt]
            lo = wid * rng
            nck = E4K[name] // SCAN

            def _issue(c, sb, db, sem, _sr=srcr, _dr=dstr):
                pltpu.async_copy(_sr.at[pl.ds(c * SCAN, SCAN)], sb, sem)
                pltpu.async_copy(_dr.at[pl.ds(c * SCAN, SCAN)], db, sem)

            def _drain(sb, db, sem, _sr=srcr, _dr=dstr):
                pltpu.make_async_copy(_sr.at[pl.ds(0, SCAN)], sb, sem).wait()
                pltpu.make_async_copy(_dr.at[pl.ds(0, SCAN)], db, sem).wait()

            def _scan(sb, db, cnt, _lo=lo, _rng=rng):
                @pl.loop(0, SCAN // 16, init_carry=cnt)
                def _s(t, c2):
                    s16 = sb[pl.ds(t * 16, 16)]
                    d16 = db[pl.ds(t * 16, 16)]
                    m = (d16 >= _lo) & (d16 < _lo + _rng)
                    plsc.store_compressed(bs.at[pl.ds(c2, 16)], s16, mask=m)
                    plsc.store_compressed(bd.at[pl.ds(c2, 16)], d16 - _lo,
                                          mask=m)
                    return c2 + plsc.all_reduce_population_count(m)[0]
                return _s

            @pl.loop(0, cap // 16)
            def _fill(i, _rng=rng):
                bs[pl.ds(i * 16, 16)] = jnp.zeros((16,), jnp.int32)
                bd[pl.ds(i * 16, 16)] = jnp.broadcast_to(
                    jnp.int32(_rng), (16,))

            _issue(0, sb0, db0, sem0)

            @pl.loop(0, nck // 2, init_carry=jnp.int32(0))
            def _outer(i, cnt, _nck=nck):
                _issue(2 * i + 1, sb1, db1, sem1)
                _drain(sb0, db0, sem0)
                cnt = _scan(sb0, db0, cnt)
                _issue(jnp.minimum(2 * i + 2, _nck - 1), sb0, db0, sem0)
                _drain(sb1, db1, sem1)
                cnt = _scan(sb1, db1, cnt)
                return cnt

            cnt = _outer
            _drain(sb0, db0, sem0)
            cbuf[pl.ds(0, 16)] = jnp.broadcast_to(cnt, (16,))
            pltpu.sync_copy(cbuf, cnt_o.at[pl.ds(wid * 16, 16)])
            pltpu.sync_copy(bs.at[pl.ds(0, cap)],
                            bsrc_o.at[pl.ds(wid * cap, cap)])
            pltpu.sync_copy(bd.at[pl.ds(0, cap)],
                            bdst_o.at[pl.ds(wid * cap, cap)])

    return k1


def _make_k2(h, bks, suffix):
    out_type = tuple(
        jax.ShapeDtypeStruct((h * NT * CAP[name],), jnp.float32)
        for name, _, _, _ in RELS)
    scratch = [
        pltpu.VMEM((ASIZE,), jnp.float32),
        pltpu.VMEM((ASIZE,), jnp.float32),
        pltpu.VMEM((656,), jnp.float32),
        pltpu.VMEM((6144,), jnp.int32),
        pltpu.VMEM((6144,), jnp.int32),
        pltpu.VMEM((2 * 6144,), jnp.float32),
        pltpu.VMEM((16,), jnp.int32),
    ]

    @functools.partial(pl.kernel, out_type=out_type, mesh=_mesh(),
                       scratch_types=scratch, compiler_params=_cparams(),
                       name="gat_edge_softmax_" + suffix)
    def k2(*refs):
        ins = refs[:30]
        atouts = refs[30:36]
        asb, adb, slab, bsb, bdb, exb, cbuf = refs[36:]
        wid = _wid()
        for ri, (name, st, dt, _e) in enumerate(RELS):
            bsrc, bdst, cnts = ins[3 * ri], ins[3 * ri + 1], ins[3 * ri + 2]
            asr, adr = ins[18 + ri], ins[24 + ri]
            cap = CAP[name]
            rng, rp1 = RNG[dt], RP1[dt]
            lo = wid * rng
            hrp = h * rp1
            atout = atouts[ri]
            pltpu.sync_copy(asr, asb.at[pl.ds(0, N_NODES[st] * h)])
            pltpu.sync_copy(adr, adb.at[pl.ds(0, N_NODES[dt] * h)])
            pltpu.sync_copy(bsrc.at[pl.ds(wid * cap, cap)],
                            bsb.at[pl.ds(0, cap)])
            pltpu.sync_copy(bdst.at[pl.ds(wid * cap, cap)],
                            bdb.at[pl.ds(0, cap)])

            @pl.loop(0, _ru(hrp, 16) // 16)
            def _zero(i):
                slab[pl.ds(i * 16, 16)] = jnp.zeros((16,), jnp.float32)

            nch = _count(cnts, wid, cbuf)

            @pl.loop(0, nch)
            def _chunk(c, _cap=cap, _lo=lo, _rp1=rp1):
                for j in range(CH // 16):
                    b = c * CH + j * 16
                    s16 = bsb[pl.ds(b, 16)]
                    d16 = bdb[pl.ds(b, 16)]
                    for hh in range(h):
                        av = plsc.load_gather(asb, [s16 * h + hh])
                        bv = plsc.load_gather(adb, [(d16 + _lo) * h + hh])
                        al = av + bv
                        al = jnp.maximum(al, al * 0.2)
                        ex = jnp.exp(al)
                        plsc.addupdate_scatter(slab, [d16 + hh * _rp1], ex)
                        exb[pl.ds(hh * _cap + b, 16)] = ex

            @pl.loop(0, _ru(hrp, 16) // 16)
            def _inv(i):
                slab[pl.ds(i * 16, 16)] = 1.0 / (slab[pl.ds(i * 16, 16)]
                                                 + 1e-16)

            @pl.loop(0, nch)
            def _attn(c, _cap=cap, _rp1=rp1):
                for j in range(CH // 16):
                    b = c * CH + j * 16
                    d16 = bdb[pl.ds(b, 16)]
                    for hh in range(h):
                        rv = plsc.load_gather(slab, [d16 + hh * _rp1])
                        exb[pl.ds(hh * _cap + b, 16)] = (
                            exb[pl.ds(hh * _cap + b, 16)] * rv)

            for hh in range(h):
                pltpu.sync_copy(
                    exb.at[pl.ds(hh * cap, cap)],
                    atout.at[pl.ds(hh * NT * cap + wid * cap, cap)])

    def run(a_src, a_dst):
        outs = k2(*(
            [x for name, _, _, _ in RELS for x in bks[name]]
            + [a_src[r[0]] for r in RELS] + [a_dst[r[0]] for r in RELS]))
        return {r[0]: outs[i] for i, r in enumerate(RELS)}

    return run


CH2 = 32


def _make_k3(h, chd, head, colw, group_rels, tbases, slab_rows, gtypes,
             suffix):
    nrel = len(group_rels)
    out_type = tuple(jax.ShapeDtypeStruct((NOUT[t], colw), jnp.float32)
                     for t in gtypes)
    scratch = [
        pltpu.VMEM((slab_rows, colw), jnp.float32),
        pltpu.VMEM((CH2, colw), jnp.float32),
        pltpu.VMEM((CH2, colw), jnp.float32),
        pltpu.VMEM((6144,), jnp.int32),
        pltpu.VMEM((6144,), jnp.int32),
        pltpu.VMEM((6144,), jnp.float32),
        pltpu.VMEM((CH2,), jnp.int32),
        pltpu.VMEM((CH2,), jnp.int32),
        pltpu.VMEM((16,), jnp.int32),
        pltpu.SemaphoreType.DMA, pltpu.SemaphoreType.DMA,
    ]

    @functools.partial(pl.kernel, out_type=out_type, mesh=_mesh(),
                       scratch_types=scratch, compiler_params=_cparams(),
                       name="gat_aggregate_" + suffix)
    def k3(*refs):
        ins = refs[:3 * nrel]
        ats = refs[3 * nrel:4 * nrel]
        hss = refs[4 * nrel:5 * nrel]
        zz = refs[5 * nrel]
        outs = refs[5 * nrel + 1:5 * nrel + 1 + len(gtypes)]
        (slab, rows0, rows1, bsb, bdb, atb, gix0, gix1, cbuf,
         sem0, sem1) = refs[5 * nrel + 1 + len(gtypes):]
        wid = _wid()

        for i in range(slab_rows // 64):
            pltpu.sync_copy(zz, slab.at[pl.ds(i * 64, 64)])
        rem = slab_rows % 64
        if rem:
            pltpu.sync_copy(zz.at[pl.ds(0, rem)],
                            slab.at[pl.ds(slab_rows - rem, rem)])

        coli = [jnp.arange(j * 16, j * 16 + 16, dtype=jnp.int32)
                for j in range(colw // 16)]

        for ri, (name, st, dt, _e) in enumerate(group_rels):
            bsrc, bdst, cnts = ins[3 * ri], ins[3 * ri + 1], ins[3 * ri + 2]
            cap = CAP[name]
            tb = tbases[dt]
            hsr = hss[ri]
            pltpu.sync_copy(bsrc.at[pl.ds(wid * cap, cap)],
                            bsb.at[pl.ds(0, cap)])
            pltpu.sync_copy(bdst.at[pl.ds(wid * cap, cap)],
                            bdb.at[pl.ds(0, cap)])
            pltpu.sync_copy(
                ats[ri].at[pl.ds(head * NT * cap + wid * cap, cap)],
                atb.at[pl.ds(0, cap)])
            pltpu.sync_copy(cnts.at[pl.ds(wid * 16, 16)], cbuf)
            cv = cbuf[pl.ds(0, 16)]
            nch = (cv[0] + CH2 - 1) // CH2

            def _prep(c, gix, rows, sem, _hs=hsr):
                base = c * CH2
                for j in range(CH2 // 16):
                    s16 = bsb[pl.ds(base + j * 16, 16)]
                    gix[pl.ds(j * 16, 16)] = s16
                pltpu.async_copy(_hs.at[gix], rows, sem)

            def _acc_chunk(c, gix, rows, sem, _hs=hsr, _tb=tb):
                pltpu.make_async_copy(_hs.at[gix], rows, sem).wait()
                base = c * CH2

                @pl.loop(0, CH2, unroll=2)
                def _acc(k):
                    k16 = jnp.broadcast_to(base + k, (16,))
                    av = plsc.load_gather(atb, [k16])
                    dlv = plsc.load_gather(bdb, [k16]) + _tb
                    for j in range(colw // 16):
                        v = rows[k, pl.ds(j * 16, 16)] * av
                        plsc.addupdate_scatter(slab, [dlv, coli[j]], v)

            @pl.loop(0, (nch + 1) // 2)
            def _pair(i):
                _prep(2 * i, gix0, rows0, sem0)
                _prep(2 * i + 1, gix1, rows1, sem1)
                _acc_chunk(2 * i, gix0, rows0, sem0)
                _acc_chunk(2 * i + 1, gix1, rows1, sem1)

        for ti, t in enumerate(gtypes):
            pltpu.sync_copy(
                slab.at[pl.ds(tbases[t], RNG[t])],
                outs[ti].at[pl.ds(wid * RNG[t], RNG[t])])

    return k3


def _proj(x, ws_list, wa, bm=512):
    n, f = x.shape
    nw = len(ws_list)
    grid = (pl.cdiv(n, bm),)

    def body(*refs):
        xr = refs[0]
        wrs = refs[1:1 + nw]
        war = refs[1 + nw]
        outs = refs[2 + nw:2 + 2 * nw]
        oa = refs[2 + 2 * nw]
        xv = xr[...]
        for wr, orf in zip(wrs, outs):
            orf[...] = jnp.dot(xv, wr[...], preferred_element_type=jnp.float32)
        oa[...] = jnp.dot(xv, war[...], preferred_element_type=jnp.float32)

    in_specs = ([pl.BlockSpec((bm, f), lambda i: (i, 0))]
                + [pl.BlockSpec((f, w.shape[1]), lambda i: (0, 0))
                   for w in ws_list]
                + [pl.BlockSpec((f, 128), lambda i: (0, 0))])
    out_specs = ([pl.BlockSpec((bm, w.shape[1]), lambda i: (i, 0))
                  for w in ws_list]
                 + [pl.BlockSpec((bm, 128), lambda i: (i, 0))])
    out_shape = ([jax.ShapeDtypeStruct((n, w.shape[1]), jnp.float32)
                  for w in ws_list]
                 + [jax.ShapeDtypeStruct((n, 128), jnp.float32)])
    return pl.pallas_call(body, grid=grid, in_specs=in_specs,
                          out_specs=out_specs, out_shape=out_shape)(
                              x, *ws_list, wa)


def _combine(parts, bias, n, width, relu, bm=512):
    npass = len(parts)
    widths = [int(q.shape[1]) for q in parts]
    offs = [sum(widths[:i]) for i in range(npass)]

    def body(*refs):
        ins = refs[:npass]
        br = refs[npass]
        orf = refs[npass + 1]
        for p in range(npass):
            o, w = offs[p], widths[p]
            v = ins[p][...] + br[0, o:o + w]
            orf[:, o:o + w] = jnp.maximum(v, 0.0) if relu else v

    in_specs = ([pl.BlockSpec((bm, w), lambda i: (i, 0)) for w in widths]
                + [pl.BlockSpec((1, width), lambda i: (0, 0))])
    return pl.pallas_call(
        body, grid=(pl.cdiv(n, bm),), in_specs=in_specs,
        out_specs=pl.BlockSpec((bm, width), lambda i: (i, 0)),
        out_shape=jax.ShapeDtypeStruct((n, width), jnp.float32))(
            *parts, bias.reshape(1, width))


def kernel(x_movie, x_user, x_genre, x_conversation, params, ei_has_genre,
           ei_has_movie, ei_rated_high, ei_rated_by, ei_mentions,
           ei_mentioned_in):
    x = {"movie": x_movie, "user": x_user, "genre": x_genre,
         "conversation": x_conversation}
    ei = {"has_genre": ei_has_genre, "has_movie": ei_has_movie,
          "rated_high": ei_rated_high, "rated_by": ei_rated_by,
          "mentions": ei_mentions, "mentioned_in": ei_mentioned_in}

    srcp, dstp = [], []
    for name, _s, _d, e in RELS:
        pad = E4K[name] - e
        srcp.append(jnp.concatenate([ei[name][0],
                                     jnp.zeros((pad,), jnp.int32)]))
        dstp.append(jnp.concatenate([ei[name][1],
                                     jnp.full((pad,), -1, jnp.int32)]))
    k1outs = _make_k1()(*(srcp + dstp))
    bks = {r[0]: k1outs[3 * i:3 * i + 3] for i, r in enumerate(RELS)}

    for l, (f_in, chd, h, concat) in enumerate(LAYERS):
        width = h * chd if concat else chd
        npass = width // 128
        lp = params[str(l)]

        wsrc, vsrc, vdst = {}, {}, {}
        for name, _s, _d, _e in RELS:
            pr = lp[name]
            wsrc[name] = pr["W_src"]
            vsrc[name] = jnp.einsum("fhc,hc->fh",
                                    pr["W_src"].reshape(f_in, h, chd),
                                    pr["att_src"])
            vdst[name] = jnp.einsum("fhc,hc->fh",
                                    pr["W_dst"].reshape(f_in, h, chd),
                                    pr["att_dst"])

        segs = ([(0, 256), (256, width - 256)] if width > 256
                else [(0, width)])
        a_src, a_dst, hs = {}, {}, {}
        for t in TYPES:
            src_rels = [r for r in RELS if r[1] == t]
            dst_rels = [r for r in RELS if r[2] == t]
            ws_list = [wsrc[r[0]][:, c0:c0 + cw]
                       for r in src_rels for (c0, cw) in segs]
            acols = ([vsrc[r[0]] for r in src_rels]
                     + [vdst[r[0]] for r in dst_rels])
            na = sum(c.shape[1] for c in acols)
            wa = jnp.concatenate(
                acols + [jnp.zeros((f_in, 128 - na), jnp.float32)], axis=1)
            outs = _proj(x[t], ws_list, wa)
            oi = 0
            for r in src_rels:
                for si in range(len(segs)):
                    hs[(r[0], si)] = outs[oi]
                    oi += 1
            ac = outs[-1]
            off = 0
            for r in src_rels:
                a_src[r[0]] = ac[:, off:off + h].reshape(-1)
                off += h
            for r in dst_rels:
                a_dst[r[0]] = ac[:, off:off + h].reshape(-1)
                off += h

        attn = _make_k2(h, bks, f"l{l}")(a_src, a_dst)

        outs_t = {}
        for g, gtypes in enumerate(GROUPS):
            rels_g = [r for r in RELS if r[2] in gtypes]
            tbases, off = {}, 0
            for t in gtypes:
                tbases[t] = off
                off += RP1[t]
            slab_rows = off
            accs = {t: [] for t in gtypes}
            for si, (c0, cw) in enumerate(segs):
                head = c0 // chd
                k3 = _make_k3(h, chd, head, cw, rels_g, tbases, slab_rows,
                              gtypes, f"l{l}g{g}s{si}")
                o = k3(*([x for r in rels_g for x in bks[r[0]]]
                         + [attn[r[0]] for r in rels_g]
                         + [hs[(r[0], si)] for r in rels_g]
                         + [jnp.zeros((64, cw), jnp.float32)]))
                if not isinstance(o, (tuple, list)):
                    o = (o,)
                for ti, t in enumerate(gtypes):
                    accs[t].append(o[ti])
            outs_t.update(accs)

        newx = {}
        for t in TYPES:
            bias_tot = sum(lp[r[0]]["bias"] for r in RELS if r[2] == t)
            newx[t] = _combine(outs_t[t], bias_tot, N_NODES[t], width,
                               relu=(l < len(LAYERS) - 1))
        x = newx

    return (x["movie"], x["user"], x["genre"], x["conversation"])


# speedup vs baseline: 1.8237x; 1.0150x over previous
import functools

import jax
import jax.numpy as jnp
from jax import lax
from jax.experimental import pallas as pl
from jax.experimental.pallas import tpu as pltpu
from jax.experimental.pallas import tpu_sc as plsc

N_NODES = {"movie": 10000, "user": 10000, "genre": 500, "conversation": 5000}
RELS = [
    ("has_genre", "movie", "genre", 30000),
    ("has_movie", "genre", "movie", 30000),
    ("rated_high", "user", "movie", 160000),
    ("rated_by", "movie", "user", 160000),
    ("mentions", "conversation", "movie", 25000),
    ("mentioned_in", "movie", "conversation", 25000),
]
LAYERS = [(128, 256, 2, True), (512, 256, 2, True), (512, 384, 1, False)]
TYPES = ["movie", "user", "genre", "conversation"]
GROUPS = [["movie"], ["user"], ["genre", "conversation"]]

NT = 32
CH = 128
SCAN = 4096


def _ru(x, m):
    return -(-x // m) * m


RNG = {t: _ru(-(-(n + 1) // NT), 8) for t, n in N_NODES.items()}
NOUT = {t: NT * r for t, r in RNG.items()}
RP1 = {t: _ru(r + 1, 8) for t, r in RNG.items()}
CAP = {"has_genre": 1280, "has_movie": 1280, "rated_high": 6144,
       "rated_by": 6144, "mentions": 1280, "mentioned_in": 1280}
E4K = {name: _ru(e, 2 * SCAN) for name, _, _, e in RELS}
ASIZE = 20608


def _mesh():
    return plsc.VectorSubcoreMesh(core_axis_name="c", subcore_axis_name="s",
                                  num_cores=2, num_subcores=16)


def _cparams():
    return pltpu.CompilerParams(needs_layout_passes=False)


def _wid():
    return lax.axis_index("c") * 16 + lax.axis_index("s")


def _count(cref, wid, buf):
    pltpu.sync_copy(cref.at[pl.ds(wid * 16, 16)], buf)
    cv = buf[pl.ds(0, 16)]
    return (cv[0] + CH - 1) // CH


def _make_k1():
    out_type = []
    for name, _s, _d, _e in RELS:
        c = CAP[name]
        out_type += [jax.ShapeDtypeStruct((NT * c,), jnp.int32),
                     jax.ShapeDtypeStruct((NT * c,), jnp.int32),
                     jax.ShapeDtypeStruct((NT * 16,), jnp.int32)]
    scratch = [
        pltpu.VMEM((6144,), jnp.int32),
        pltpu.VMEM((6144,), jnp.int32),
        pltpu.VMEM((SCAN,), jnp.int32), pltpu.VMEM((SCAN,), jnp.int32),
        pltpu.VMEM((SCAN,), jnp.int32), pltpu.VMEM((SCAN,), jnp.int32),
        pltpu.VMEM((16,), jnp.int32),
        pltpu.SemaphoreType.DMA, pltpu.SemaphoreType.DMA,
    ]

    @functools.partial(pl.kernel, out_type=tuple(out_type), mesh=_mesh(),
                       scratch_types=scratch, compiler_params=_cparams(),
                       name="gat_bucket_edges")
    def k1(*refs):
        ins = refs[:12]
        outs = refs[12:30]
        bs, bd, sb0, db0, sb1, db1, cbuf, sem0, sem1 = refs[30:]
        wid = _wid()
        for ri, (name, _st, dt, _e) in enumerate(RELS):
            srcr, dstr = ins[ri], ins[6 + ri]
            bsrc_o, bdst_o, cnt_o = (outs[3 * ri], outs[3 * ri + 1],
                                     outs[3 * ri + 2])
            cap = CAP[name]
            rng = RNG[dt]
            lo = wid * rng
            nck = E4K[name] // SCAN

            def _issue(c, sb, db, sem, _sr=srcr, _dr=dstr):
                pltpu.async_copy(_sr.at[pl.ds(c * SCAN, SCAN)], sb, sem)
                pltpu.async_copy(_dr.at[pl.ds(c * SCAN, SCAN)], db, sem)

            def _drain(sb, db, sem, _sr=srcr, _dr=dstr):
                pltpu.make_async_copy(_sr.at[pl.ds(0, SCAN)], sb, sem).wait()
                pltpu.make_async_copy(_dr.at[pl.ds(0, SCAN)], db, sem).wait()

            def _scan(sb, db, cnt, _lo=lo, _rng=rng):
                @pl.loop(0, SCAN // 16, init_carry=cnt)
                def _s(t, c2):
                    s16 = sb[pl.ds(t * 16, 16)]
                    d16 = db[pl.ds(t * 16, 16)]
                    m = (d16 >= _lo) & (d16 < _lo + _rng)
                    plsc.store_compressed(bs.at[pl.ds(c2, 16)], s16, mask=m)
                    plsc.store_compressed(bd.at[pl.ds(c2, 16)], d16 - _lo,
                                          mask=m)
                    return c2 + plsc.all_reduce_population_count(m)[0]
                return _s

            @pl.loop(0, cap // 16)
            def _fill(i, _rng=rng):
                bs[pl.ds(i * 16, 16)] = jnp.zeros((16,), jnp.int32)
                bd[pl.ds(i * 16, 16)] = jnp.broadcast_to(
                    jnp.int32(_rng), (16,))

            _issue(0, sb0, db0, sem0)

            @pl.loop(0, nck // 2, init_carry=jnp.int32(0))
            def _outer(i, cnt, _nck=nck):
                _issue(2 * i + 1, sb1, db1, sem1)
                _drain(sb0, db0, sem0)
                cnt = _scan(sb0, db0, cnt)
                _issue(jnp.minimum(2 * i + 2, _nck - 1), sb0, db0, sem0)
                _drain(sb1, db1, sem1)
                cnt = _scan(sb1, db1, cnt)
                return cnt

            cnt = _outer
            _drain(sb0, db0, sem0)
            cbuf[pl.ds(0, 16)] = jnp.broadcast_to(cnt, (16,))
            pltpu.sync_copy(cbuf, cnt_o.at[pl.ds(wid * 16, 16)])
            pltpu.sync_copy(bs.at[pl.ds(0, cap)],
                            bsrc_o.at[pl.ds(wid * cap, cap)])
            pltpu.sync_copy(bd.at[pl.ds(0, cap)],
                            bdst_o.at[pl.ds(wid * cap, cap)])

    return k1


def _make_k2(h, bks, suffix):
    out_type = tuple(
        jax.ShapeDtypeStruct((h * NT * CAP[name],), jnp.float32)
        for name, _, _, _ in RELS)
    scratch = [
        pltpu.VMEM((ASIZE,), jnp.float32),
        pltpu.VMEM((ASIZE,), jnp.float32),
        pltpu.VMEM((656,), jnp.float32),
        pltpu.VMEM((6144,), jnp.int32),
        pltpu.VMEM((6144,), jnp.int32),
        pltpu.VMEM((2 * 6144,), jnp.float32),
        pltpu.VMEM((16,), jnp.int32),
    ]

    @functools.partial(pl.kernel, out_type=out_type, mesh=_mesh(),
                       scratch_types=scratch, compiler_params=_cparams(),
                       name="gat_edge_softmax_" + suffix)
    def k2(*refs):
        ins = refs[:30]
        atouts = refs[30:36]
        asb, adb, slab, bsb, bdb, exb, cbuf = refs[36:]
        wid = _wid()
        for ri, (name, st, dt, _e) in enumerate(RELS):
            bsrc, bdst, cnts = ins[3 * ri], ins[3 * ri + 1], ins[3 * ri + 2]
            asr, adr = ins[18 + ri], ins[24 + ri]
            cap = CAP[name]
            rng, rp1 = RNG[dt], RP1[dt]
            lo = wid * rng
            hrp = h * rp1
            atout = atouts[ri]
            pltpu.sync_copy(asr, asb.at[pl.ds(0, N_NODES[st] * h)])
            pltpu.sync_copy(adr, adb.at[pl.ds(0, N_NODES[dt] * h)])
            pltpu.sync_copy(bsrc.at[pl.ds(wid * cap, cap)],
                            bsb.at[pl.ds(0, cap)])
            pltpu.sync_copy(bdst.at[pl.ds(wid * cap, cap)],
                            bdb.at[pl.ds(0, cap)])

            @pl.loop(0, _ru(hrp, 16) // 16)
            def _zero(i):
                slab[pl.ds(i * 16, 16)] = jnp.zeros((16,), jnp.float32)

            nch = _count(cnts, wid, cbuf)

            @pl.loop(0, nch)
            def _chunk(c, _cap=cap, _lo=lo, _rp1=rp1):
                for j in range(CH // 16):
                    b = c * CH + j * 16
                    s16 = bsb[pl.ds(b, 16)]
                    d16 = bdb[pl.ds(b, 16)]
                    for hh in range(h):
                        av = plsc.load_gather(asb, [s16 * h + hh])
                        bv = plsc.load_gather(adb, [(d16 + _lo) * h + hh])
                        al = av + bv
                        al = jnp.maximum(al, al * 0.2)
                        ex = jnp.exp(al)
                        plsc.addupdate_scatter(slab, [d16 + hh * _rp1], ex)
                        exb[pl.ds(hh * _cap + b, 16)] = ex

            @pl.loop(0, _ru(hrp, 16) // 16)
            def _inv(i):
                slab[pl.ds(i * 16, 16)] = 1.0 / (slab[pl.ds(i * 16, 16)]
                                                 + 1e-16)

            @pl.loop(0, nch)
            def _attn(c, _cap=cap, _rp1=rp1):
                for j in range(CH // 16):
                    b = c * CH + j * 16
                    d16 = bdb[pl.ds(b, 16)]
                    for hh in range(h):
                        rv = plsc.load_gather(slab, [d16 + hh * _rp1])
                        exb[pl.ds(hh * _cap + b, 16)] = (
                            exb[pl.ds(hh * _cap + b, 16)] * rv)

            for hh in range(h):
                pltpu.sync_copy(
                    exb.at[pl.ds(hh * cap, cap)],
                    atout.at[pl.ds(hh * NT * cap + wid * cap, cap)])

    def run(a_src, a_dst):
        outs = k2(*(
            [x for name, _, _, _ in RELS for x in bks[name]]
            + [a_src[r[0]] for r in RELS] + [a_dst[r[0]] for r in RELS]))
        return {r[0]: outs[i] for i, r in enumerate(RELS)}

    return run


CH2 = 32


def _make_k3(h, chd, head, colw, group_rels, tbases, slab_rows, gtypes,
             suffix):
    nrel = len(group_rels)
    out_type = tuple(jax.ShapeDtypeStruct((NOUT[t], colw), jnp.float32)
                     for t in gtypes)
    scratch = [
        pltpu.VMEM((slab_rows, colw), jnp.float32),
        pltpu.VMEM((CH2, colw), jnp.float32),
        pltpu.VMEM((CH2, colw), jnp.float32),
        pltpu.VMEM((6144,), jnp.int32),
        pltpu.VMEM((6144,), jnp.int32),
        pltpu.VMEM((6144,), jnp.float32),
        pltpu.VMEM((CH2,), jnp.int32),
        pltpu.VMEM((CH2,), jnp.int32),
        pltpu.VMEM((16,), jnp.int32),
        pltpu.SemaphoreType.DMA, pltpu.SemaphoreType.DMA,
    ]

    @functools.partial(pl.kernel, out_type=out_type, mesh=_mesh(),
                       scratch_types=scratch, compiler_params=_cparams(),
                       name="gat_aggregate_" + suffix)
    def k3(*refs):
        ins = refs[:3 * nrel]
        ats = refs[3 * nrel:4 * nrel]
        hss = refs[4 * nrel:5 * nrel]
        zz = refs[5 * nrel]
        outs = refs[5 * nrel + 1:5 * nrel + 1 + len(gtypes)]
        (slab, rows0, rows1, bsb, bdb, atb, gix0, gix1, cbuf,
         sem0, sem1) = refs[5 * nrel + 1 + len(gtypes):]
        wid = _wid()

        for i in range(slab_rows // 64):
            pltpu.sync_copy(zz, slab.at[pl.ds(i * 64, 64)])
        rem = slab_rows % 64
        if rem:
            pltpu.sync_copy(zz.at[pl.ds(0, rem)],
                            slab.at[pl.ds(slab_rows - rem, rem)])

        coli = [jnp.arange(j * 16, j * 16 + 16, dtype=jnp.int32)
                for j in range(colw // 16)]

        for ri, (name, st, dt, _e) in enumerate(group_rels):
            bsrc, bdst, cnts = ins[3 * ri], ins[3 * ri + 1], ins[3 * ri + 2]
            cap = CAP[name]
            tb = tbases[dt]
            hsr = hss[ri]
            pltpu.sync_copy(bsrc.at[pl.ds(wid * cap, cap)],
                            bsb.at[pl.ds(0, cap)])
            pltpu.sync_copy(bdst.at[pl.ds(wid * cap, cap)],
                            bdb.at[pl.ds(0, cap)])
            pltpu.sync_copy(
                ats[ri].at[pl.ds(head * NT * cap + wid * cap, cap)],
                atb.at[pl.ds(0, cap)])
            pltpu.sync_copy(cnts.at[pl.ds(wid * 16, 16)], cbuf)
            cv = cbuf[pl.ds(0, 16)]
            nch = (cv[0] + CH2 - 1) // CH2

            def _prep(c, gix, rows, sem, _hs=hsr):
                base = c * CH2
                for j in range(CH2 // 16):
                    s16 = bsb[pl.ds(base + j * 16, 16)]
                    gix[pl.ds(j * 16, 16)] = s16
                pltpu.async_copy(_hs.at[gix], rows, sem)

            def _acc_chunk(c, gix, rows, sem, _hs=hsr, _tb=tb):
                pltpu.make_async_copy(_hs.at[gix], rows, sem).wait()
                base = c * CH2

                @pl.loop(0, CH2, unroll=2)
                def _acc(k):
                    k16 = jnp.broadcast_to(base + k, (16,))
                    av = plsc.load_gather(atb, [k16])
                    dlv = plsc.load_gather(bdb, [k16]) + _tb
                    for j in range(colw // 16):
                        v = rows[k, pl.ds(j * 16, 16)] * av
                        plsc.addupdate_scatter(slab, [dlv, coli[j]], v)

            @pl.loop(0, (nch + 1) // 2)
            def _pair(i):
                _prep(2 * i, gix0, rows0, sem0)
                _prep(2 * i + 1, gix1, rows1, sem1)
                _acc_chunk(2 * i, gix0, rows0, sem0)
                _acc_chunk(2 * i + 1, gix1, rows1, sem1)

        for ti, t in enumerate(gtypes):
            pltpu.sync_copy(
                slab.at[pl.ds(tbases[t], RNG[t])],
                outs[ti].at[pl.ds(wid * RNG[t], RNG[t])])

    return k3


def _proj(x, ws_list, wa, bm=512):
    n, f = x.shape
    nw = len(ws_list)
    grid = (pl.cdiv(n, bm),)

    def body(*refs):
        xr = refs[0]
        wrs = refs[1:1 + nw]
        war = refs[1 + nw]
        outs = refs[2 + nw:2 + 2 * nw]
        oa = refs[2 + 2 * nw]
        xv = xr[...]
        for wr, orf in zip(wrs, outs):
            orf[...] = jnp.dot(xv, wr[...], preferred_element_type=jnp.float32)
        oa[...] = jnp.dot(xv, war[...], preferred_element_type=jnp.float32)

    in_specs = ([pl.BlockSpec((bm, f), lambda i: (i, 0))]
                + [pl.BlockSpec((f, w.shape[1]), lambda i: (0, 0))
                   for w in ws_list]
                + [pl.BlockSpec((f, 128), lambda i: (0, 0))])
    out_specs = ([pl.BlockSpec((bm, w.shape[1]), lambda i: (i, 0))
                  for w in ws_list]
                 + [pl.BlockSpec((bm, 128), lambda i: (i, 0))])
    out_shape = ([jax.ShapeDtypeStruct((n, w.shape[1]), jnp.float32)
                  for w in ws_list]
                 + [jax.ShapeDtypeStruct((n, 128), jnp.float32)])
    return pl.pallas_call(body, grid=grid, in_specs=in_specs,
                          out_specs=out_specs, out_shape=out_shape)(
                              x, *ws_list, wa)


def _combine(parts, bias, n, width, relu, bm=512):
    npass = len(parts)
    widths = [int(q.shape[1]) for q in parts]
    offs = [sum(widths[:i]) for i in range(npass)]

    def body(*refs):
        ins = refs[:npass]
        br = refs[npass]
        orf = refs[npass + 1]
        for p in range(npass):
            o, w = offs[p], widths[p]
            v = ins[p][...] + br[0, o:o + w]
            orf[:, o:o + w] = jnp.maximum(v, 0.0) if relu else v

    in_specs = ([pl.BlockSpec((bm, w), lambda i: (i, 0)) for w in widths]
                + [pl.BlockSpec((1, width), lambda i: (0, 0))])
    return pl.pallas_call(
        body, grid=(pl.cdiv(n, bm),), in_specs=in_specs,
        out_specs=pl.BlockSpec((bm, width), lambda i: (i, 0)),
        out_shape=jax.ShapeDtypeStruct((n, width), jnp.float32))(
            *parts, bias.reshape(1, width))


def kernel(x_movie, x_user, x_genre, x_conversation, params, ei_has_genre,
           ei_has_movie, ei_rated_high, ei_rated_by, ei_mentions,
           ei_mentioned_in):
    x = {"movie": x_movie, "user": x_user, "genre": x_genre,
         "conversation": x_conversation}
    ei = {"has_genre": ei_has_genre, "has_movie": ei_has_movie,
          "rated_high": ei_rated_high, "rated_by": ei_rated_by,
          "mentions": ei_mentions, "mentioned_in": ei_mentioned_in}

    srcp, dstp = [], []
    for name, _s, _d, e in RELS:
        pad = E4K[name] - e
        srcp.append(jnp.concatenate([ei[name][0],
                                     jnp.zeros((pad,), jnp.int32)]))
        dstp.append(jnp.concatenate([ei[name][1],
                                     jnp.full((pad,), -1, jnp.int32)]))
    k1outs = _make_k1()(*(srcp + dstp))
    bks = {r[0]: k1outs[3 * i:3 * i + 3] for i, r in enumerate(RELS)}

    for l, (f_in, chd, h, concat) in enumerate(LAYERS):
        width = h * chd if concat else chd
        npass = width // 128
        lp = params[str(l)]

        wsrc, vsrc, vdst = {}, {}, {}
        for name, _s, _d, _e in RELS:
            pr = lp[name]
            wsrc[name] = pr["W_src"]
            vsrc[name] = jnp.einsum("fhc,hc->fh",
                                    pr["W_src"].reshape(f_in, h, chd),
                                    pr["att_src"])
            vdst[name] = jnp.einsum("fhc,hc->fh",
                                    pr["W_dst"].reshape(f_in, h, chd),
                                    pr["att_dst"])

        segs = ([(0, 256), (256, width - 256)] if width > 256
                else [(0, width)])
        a_src, a_dst, hs = {}, {}, {}
        for t in TYPES:
            src_rels = [r for r in RELS if r[1] == t]
            dst_rels = [r for r in RELS if r[2] == t]
            ws_list = [wsrc[r[0]][:, c0:c0 + cw]
                       for r in src_rels for (c0, cw) in segs]
            acols = ([vsrc[r[0]] for r in src_rels]
                     + [vdst[r[0]] for r in dst_rels])
            na = sum(c.shape[1] for c in acols)
            wa = jnp.concatenate(
                acols + [jnp.zeros((f_in, 128 - na), jnp.float32)], axis=1)
            outs = _proj(x[t], ws_list, wa)
            oi = 0
            for r in src_rels:
                for si in range(len(segs)):
                    hs[(r[0], si)] = outs[oi]
                    oi += 1
            ac = outs[-1]
            off = 0
            for r in src_rels:
                a_src[r[0]] = ac[:, off:off + h].reshape(-1)
                off += h
            for r in dst_rels:
                a_dst[r[0]] = ac[:, off:off + h].reshape(-1)
                off += h

        attn = _make_k2(h, bks, f"l{l}")(a_src, a_dst)

        outs_t = {}
        for g, gtypes in enumerate(GROUPS):
            rels_g = [r for r in RELS if r[2] in gtypes]
            tbases, off = {}, 0
            for t in gtypes:
                tbases[t] = off
                off += RP1[t]
            slab_rows = off
            accs = {t: [] for t in gtypes}
            for si, (c0, cw) in enumerate(segs):
                head = c0 // chd
                k3 = _make_k3(h, chd, head, cw, rels_g, tbases, slab_rows,
                              gtypes, f"l{l}g{g}s{si}")
                o = k3(*([x for r in rels_g for x in bks[r[0]]]
                         + [attn[r[0]] for r in rels_g]
                         + [hs[(r[0], si)] for r in rels_g]
                         + [jnp.zeros((64, cw), jnp.float32)]))
                if not isinstance(o, (tuple, list)):
                    o = (o,)
                for ti, t in enumerate(gtypes):
                    accs[t].append(o[ti])
            outs_t.update(accs)

        newx = {}
        for t in TYPES:
            bias_tot = sum(lp[r[0]]["bias"] for r in RELS if r[2] == t)
            newx[t] = _combine(outs_t[t], bias_tot, N_NODES[t], width,
                               relu=(l < len(LAYERS) - 1))
        x = newx

    return (x["movie"], x["user"], x["genre"], x["conversation"])
